# baseline (speedup 1.0000x reference)
import numpy as np
import jax
import jax.numpy as jnp
from jax.experimental import pallas as pl
from jax.experimental.pallas import tpu as pltpu

_NUM_CLASSES = 4
_IMAGE_SHAPE = (256, 256)
_SCORE_THR = 0.05
_SIGMA = 0.05
_MAX_PER_CLASS = 100
_MAX_DET = 100

_B = 2
_LANE = 128
_NEG = -jnp.inf


def _gen_anchors(image_shape):
    aspect_ratios = [0.5, 1.0, 2.0]
    scales = [2.0 ** x for x in [0.0, 1.0 / 3.0, 2.0 / 3.0]]
    areas = [float(x) ** 2 for x in [32, 64, 128, 256, 512]]
    all_anchors = []
    for level, area in zip(range(3, 8), areas):
        stride = 2 ** level
        dims = []
        for ratio in aspect_ratios:
            h = np.sqrt(area / ratio)
            w = area / h
            for s in scales:
                dims.append([w * s, h * s])
        dims = np.asarray(dims, np.float32)
        fh = int(np.ceil(image_shape[0] / stride))
        fw = int(np.ceil(image_shape[1] / stride))
        cx = (np.arange(fw, dtype=np.float32) + 0.5) * stride
        cy = (np.arange(fh, dtype=np.float32) + 0.5) * stride
        cxg, cyg = np.meshgrid(cx, cy)
        centers = np.stack([cxg, cyg], axis=-1).reshape(-1, 1, 2)
        centers = np.tile(centers, (1, dims.shape[0], 1))
        d = np.tile(dims[None, :, :], (centers.shape[0], 1, 1))
        all_anchors.append(np.concatenate([centers, d], axis=-1).reshape(-1, 4))
    return np.concatenate(all_anchors, axis=0)


_ANCHORS_NP = _gen_anchors(_IMAGE_SHAPE)
_N = _ANCHORS_NP.shape[0]
_NPAD = ((_N + _LANE - 1) // _LANE) * _LANE


def _nms_body(box_ref, cls_ref, anc_ref, boxT_ref, clsT_ref, ancT_ref,
              vd_ref, bx_ref, sc_ref, cl_ref, s_ref, ct_ref, ix_ref):
    f32 = jnp.float32
    rowid = jax.lax.broadcasted_iota(jnp.int32, (8, _NPAD), 0)
    li = jax.lax.broadcasted_iota(jnp.int32, (8, _NPAD), 1)
    ki = jax.lax.broadcasted_iota(jnp.int32, (8, _LANE), 1)

    acx = anc_ref[0:1, :]
    acy = anc_ref[1:2, :]
    aw = anc_ref[2:3, :]
    ah = anc_ref[3:4, :]

    coords = []
    for b in range(_B):
        tx = box_ref[b, 0:1, :]
        ty = box_ref[b, 1:2, :]
        tw = box_ref[b, 2:3, :]
        th = box_ref[b, 3:4, :]
        cx = tx * aw + acx
        cy = ty * ah + acy
        w = jnp.exp(tw) * aw
        h = jnp.exp(th) * ah
        x1 = cx - w / 2.0
        y1 = cy - h / 2.0
        x2 = cx + w / 2.0
        y2 = cy + h / 2.0
        ar = (x2 - x1) * (y2 - y1)
        coords.append((x1, y1, x2, y2, ar))

    def expand(v0, v1):
        return jnp.where(rowid < 4, v0, v1)

    x1a = expand(coords[0][0], coords[1][0])
    y1a = expand(coords[0][1], coords[1][1])
    x2a = expand(coords[0][2], coords[1][2])
    y2a = expand(coords[0][3], coords[1][3])
    areas8 = expand(coords[0][4], coords[1][4])

    awh = ancT_ref[:, 2:4]
    axy = ancT_ref[:, 0:2]
    for b in range(_B):
        cxy = boxT_ref[:, 4 * b: 4 * b + 2] * awh + axy
        wh = jnp.exp(boxT_ref[:, 4 * b + 2: 4 * b + 4]) * awh
        half = wh * 0.5
        ct_ref[:, 8 * b: 8 * b + 2] = cxy - half
        ct_ref[:, 8 * b + 2: 8 * b + 4] = cxy + half
        sig = jax.nn.sigmoid(clsT_ref[:, 4 * b: 4 * b + 4])
        best = sig[:, 0:1]
        bidx = jnp.zeros_like(best)
        for c in range(1, 4):
            sc = sig[:, c: c + 1]
            gt = sc > best
            bidx = jnp.where(gt, f32(c), bidx)
            best = jnp.maximum(sc, best)
        ct_ref[:, 8 * b + 4: 8 * b + 5] = bidx

    scores0 = jax.nn.sigmoid(cls_ref[...])
    s_ref[...] = jnp.where(scores0 > _SCORE_THR, scores0, _NEG)

    def step(t, carry):
        acc_s, ax1, ay1, ax2, ay2, acls, aval = carry
        s = s_ref[...]
        m = jnp.max(s, axis=1, keepdims=True)
        eqm = s == m
        idxv = jnp.min(jnp.where(eqm, li, _NPAD), axis=1, keepdims=True)
        ix_ref[:, 0:1] = idxv
        rows = []
        for r in range(8):
            b = r // 4
            ir = ix_ref[r, 0]
            rows.append(ct_ref[pl.ds(ir, 1), 8 * b: 8 * b + 5])
        L = jnp.concatenate(rows, axis=0)
        bx1 = L[:, 0:1]
        by1 = L[:, 1:2]
        bx2 = L[:, 2:3]
        by2 = L[:, 3:4]
        bcl = L[:, 4:5]
        valid = m > _SCORE_THR
        validf = valid.astype(f32)

        xx1 = jnp.maximum(bx1, x1a)
        yy1 = jnp.maximum(by1, y1a)
        xx2 = jnp.minimum(bx2, x2a)
        yy2 = jnp.minimum(by2, y2a)
        inter = jnp.maximum(xx2 - xx1, 0.0) * jnp.maximum(yy2 - yy1, 0.0)
        a_i = (bx2 - bx1) * (by2 - by1)
        iou = inter / (a_i + areas8 - inter + 1e-8)
        weight = jnp.exp(-0.5 * iou * iou / _SIGMA)
        ns = s * weight
        onehot = li == idxv
        keep = (ns > _SCORE_THR) & jnp.logical_not(onehot)
        s_ref[...] = jnp.where(keep, ns, _NEG)

        colhot = (ki == t).astype(f32)
        ssel = jnp.where(valid, m, 0.0)
        acc_s = acc_s + colhot * (ssel * validf)
        ax1 = ax1 + colhot * (bx1 * validf)
        ay1 = ay1 + colhot * (by1 * validf)
        ax2 = ax2 + colhot * (bx2 * validf)
        ay2 = ay2 + colhot * (by2 * validf)
        acls = acls + colhot * (bcl * validf)
        aval = aval + colhot * validf
        return acc_s, ax1, ay1, ax2, ay2, acls, aval

    zeros8 = jnp.zeros((8, _LANE), f32)
    acc_s, ax1, ay1, ax2, ay2, acls, aval = jax.lax.fori_loop(
        0, _MAX_PER_CLASS, step,
        (zeros8, zeros8, zeros8, zeros8, zeros8, zeros8, zeros8))

    okey = (jax.lax.broadcasted_iota(jnp.int32, (4, _LANE), 0) * _LANE
            + jax.lax.broadcasted_iota(jnp.int32, (4, _LANE), 1))
    kl = jax.lax.broadcasted_iota(jnp.int32, (1, _LANE), 1)
    BIG = jnp.int32(1 << 20)

    for b in range(_B):
        r0, r1 = 4 * b, 4 * b + 4
        v4 = aval[r0:r1] > 0.0
        s4 = acc_s[r0:r1]
        x14 = ax1[r0:r1]
        y14 = ay1[r0:r1]
        x24 = ax2[r0:r1]
        y24 = ay2[r0:r1]
        c4 = acls[r0:r1]
        nv = jnp.sum(v4.astype(jnp.int32), axis=(0, 1), keepdims=True)

        def sel_sums(hot):
            def red(v):
                return jnp.sum(hot * v, axis=(0, 1), keepdims=True)
            return red(x14), red(y14), red(x24), red(y24), red(s4), red(c4)

        def cstep(k, carry):
            taken, ox1, oy1, ox2, oy2, osc, ocl = carry
            mask = v4 & (taken > 0.0)
            key = jnp.where(mask, okey, BIG)
            mk = jnp.min(key, axis=(0, 1), keepdims=True)
            sel = (okey == mk) & mask
            hot = sel.astype(jnp.float32)
            vx1, vy1, vx2, vy2, vsc, vcl = sel_sums(hot)
            khot = (kl == k).astype(jnp.float32)
            ox1 = ox1 + khot * vx1
            oy1 = oy1 + khot * vy1
            ox2 = ox2 + khot * vx2
            oy2 = oy2 + khot * vy2
            osc = osc + khot * vsc
            ocl = ocl + khot * vcl
            taken = jnp.where(sel, 0.0, taken)
            return taken, ox1, oy1, ox2, oy2, osc, ocl

        z1 = jnp.zeros((1, _LANE), jnp.float32)
        ones4 = jnp.ones((4, _LANE), jnp.float32)
        _, ox1, oy1, ox2, oy2, osc, ocl = jax.lax.fori_loop(
            0, _MAX_DET, cstep, (ones4, z1, z1, z1, z1, z1, z1))

        def tstep(k, carry):
            taken, px1, py1, px2, py2, psc = carry
            mask = taken > 0.0
            sc = jnp.where(mask, s4, -1.0)
            ms = jnp.max(sc, axis=(0, 1), keepdims=True)
            kk = jnp.where(sc == ms, okey, BIG)
            mkk = jnp.min(kk, axis=(0, 1), keepdims=True)
            sel = okey == mkk
            hot = sel.astype(jnp.float32)
            vx1, vy1, vx2, vy2, _, _ = sel_sums(hot)
            khot = (kl == k).astype(jnp.float32)
            px1 = px1 + khot * vx1
            py1 = py1 + khot * vy1
            px2 = px2 + khot * vx2
            py2 = py2 + khot * vy2
            psc = psc + khot * ms
            taken = jnp.where(sel, 0.0, taken)
            return taken, px1, py1, px2, py2, psc

        _, px1, py1, px2, py2, psc = jax.lax.fori_loop(
            0, _MAX_DET, tstep, (ones4, z1, z1, z1, z1, z1))

        def lane_val(vec, j):
            return jnp.sum(jnp.where(kl == j, vec, 0.0), axis=(0, 1),
                           keepdims=True)
        cc0 = lane_val(ocl, 0)
        cc1 = lane_val(ocl, 1)
        cc2 = lane_val(ocl, 2)
        cc3 = lane_val(ocl, 3)
        buggy = jnp.where(ocl == 0.0, cc0,
                          jnp.where(ocl == 1.0, cc1,
                                    jnp.where(ocl == 2.0, cc2, cc3)))

        use_keep = nv <= _MAX_DET
        fx1 = jnp.where(use_keep, ox1, px1)
        fy1 = jnp.where(use_keep, oy1, py1)
        fx2 = jnp.where(use_keep, ox2, px2)
        fy2 = jnp.where(use_keep, oy2, py2)
        fsc = jnp.where(use_keep, osc, psc)
        ckeep = jnp.where(kl < nv, ocl, -1.0)
        fcl = jnp.where(use_keep, ckeep, buggy)

        bx_ref[b] = jnp.concatenate([fx1, fy1, fx2, fy2], axis=0)
        sc_ref[b] = fsc
        cl_ref[b] = fcl.astype(jnp.int32)
        vd_ref[b] = jnp.broadcast_to(jnp.minimum(nv, _MAX_DET), (1, _LANE))


def kernel(predictions):
    p = predictions.astype(jnp.float32)
    box_t = jnp.transpose(p[:, :, :4], (0, 2, 1))
    box_t = jnp.pad(box_t, ((0, 0), (0, 0), (0, _NPAD - _N)))
    cls_t = jnp.transpose(p[:, :, 4:], (0, 2, 1)).reshape(8, _N)
    cls_t = jnp.pad(cls_t, ((0, 0), (0, _NPAD - _N)),
                    constant_values=-1e30)
    anc = jnp.asarray(_ANCHORS_NP.T, jnp.float32)
    anc = jnp.pad(anc, ((0, 0), (0, _NPAD - _N)))

    boxT = jnp.concatenate([p[0, :, :4], p[1, :, :4]], axis=1)
    boxT = jnp.pad(boxT, ((0, _NPAD - _N), (0, 0)))
    clsT = jnp.concatenate([p[0, :, 4:], p[1, :, 4:]], axis=1)
    clsT = jnp.pad(clsT, ((0, _NPAD - _N), (0, 0)))
    ancT = jnp.asarray(_ANCHORS_NP, jnp.float32)
    ancT = jnp.pad(ancT, ((0, _NPAD - _N), (0, 0)), constant_values=1.0)

    out_shape = [
        jax.ShapeDtypeStruct((_B, 1, _LANE), jnp.int32),
        jax.ShapeDtypeStruct((_B, 4, _LANE), jnp.float32),
        jax.ShapeDtypeStruct((_B, 1, _LANE), jnp.float32),
        jax.ShapeDtypeStruct((_B, 1, _LANE), jnp.int32),
    ]
    vd, bx, sc, cl = pl.pallas_call(
        _nms_body,
        out_shape=out_shape,
        scratch_shapes=[
            pltpu.VMEM((8, _NPAD), jnp.float32),
            pltpu.VMEM((_NPAD, 16), jnp.float32),
            pltpu.VMEM((8, _LANE), jnp.int32),
        ],
    )(box_t, cls_t, anc, boxT, clsT, ancT)

    valid_detections = vd[:, 0, 0]
    nmsed_boxes = jnp.transpose(bx, (0, 2, 1))[:, :_MAX_DET, :]
    nmsed_scores = sc[:, 0, :_MAX_DET]
    nmsed_classes = cl[:, 0, :_MAX_DET]
    return valid_detections, nmsed_boxes, nmsed_scores, nmsed_classes


# speedup vs baseline: 1.1766x; 1.1766x over previous
import numpy as np
import jax
import jax.numpy as jnp
from jax.experimental import pallas as pl
from jax.experimental.pallas import tpu as pltpu

_NUM_CLASSES = 4
_IMAGE_SHAPE = (256, 256)
_SCORE_THR = 0.05
_SIGMA = 0.05
_MAX_PER_CLASS = 100
_MAX_DET = 100

_B = 2
_LANE = 128
_NEG = -jnp.inf


def _gen_anchors(image_shape):
    aspect_ratios = [0.5, 1.0, 2.0]
    scales = [2.0 ** x for x in [0.0, 1.0 / 3.0, 2.0 / 3.0]]
    areas = [float(x) ** 2 for x in [32, 64, 128, 256, 512]]
    all_anchors = []
    for level, area in zip(range(3, 8), areas):
        stride = 2 ** level
        dims = []
        for ratio in aspect_ratios:
            h = np.sqrt(area / ratio)
            w = area / h
            for s in scales:
                dims.append([w * s, h * s])
        dims = np.asarray(dims, np.float32)
        fh = int(np.ceil(image_shape[0] / stride))
        fw = int(np.ceil(image_shape[1] / stride))
        cx = (np.arange(fw, dtype=np.float32) + 0.5) * stride
        cy = (np.arange(fh, dtype=np.float32) + 0.5) * stride
        cxg, cyg = np.meshgrid(cx, cy)
        centers = np.stack([cxg, cyg], axis=-1).reshape(-1, 1, 2)
        centers = np.tile(centers, (1, dims.shape[0], 1))
        d = np.tile(dims[None, :, :], (centers.shape[0], 1, 1))
        all_anchors.append(np.concatenate([centers, d], axis=-1).reshape(-1, 4))
    return np.concatenate(all_anchors, axis=0)


_ANCHORS_NP = _gen_anchors(_IMAGE_SHAPE)
_N = _ANCHORS_NP.shape[0]
_NPAD = ((_N + _LANE - 1) // _LANE) * _LANE


def _nms_body(box_ref, cls_ref, anc_ref,
              vd_ref, bx_ref, sc_ref, cl_ref, s_ref):
    f32 = jnp.float32
    rowid = jax.lax.broadcasted_iota(jnp.int32, (8, _NPAD), 0)
    li = jax.lax.broadcasted_iota(jnp.int32, (8, _NPAD), 1)
    ki = jax.lax.broadcasted_iota(jnp.int32, (8, _LANE), 1)

    acx = anc_ref[0:1, :]
    acy = anc_ref[1:2, :]
    aw = anc_ref[2:3, :]
    ah = anc_ref[3:4, :]

    coords = []
    for b in range(_B):
        tx = box_ref[b, 0:1, :]
        ty = box_ref[b, 1:2, :]
        tw = box_ref[b, 2:3, :]
        th = box_ref[b, 3:4, :]
        cx = tx * aw + acx
        cy = ty * ah + acy
        w = jnp.exp(tw) * aw
        h = jnp.exp(th) * ah
        x1 = cx - w / 2.0
        y1 = cy - h / 2.0
        x2 = cx + w / 2.0
        y2 = cy + h / 2.0
        ar = (x2 - x1) * (y2 - y1)
        coords.append((x1, y1, x2, y2, ar))

    def expand(v0, v1):
        return jnp.where(rowid < 4, v0, v1)

    x1a = expand(coords[0][0], coords[1][0])
    y1a = expand(coords[0][1], coords[1][1])
    x2a = expand(coords[0][2], coords[1][2])
    y2a = expand(coords[0][3], coords[1][3])
    areas8 = expand(coords[0][4], coords[1][4])

    scores0 = jax.nn.sigmoid(cls_ref[...])
    s_ref[...] = jnp.where(scores0 > _SCORE_THR, scores0, _NEG)

    def batch_cls(b):
        best = scores0[4 * b: 4 * b + 1, :]
        bidx = jnp.zeros_like(best)
        for c in range(1, 4):
            sc = scores0[4 * b + c: 4 * b + c + 1, :]
            gt = sc > best
            bidx = jnp.where(gt, f32(c), bidx)
            best = jnp.maximum(sc, best)
        return bidx
    cls8 = jnp.where(rowid < 4,
                     jnp.broadcast_to(batch_cls(0), (8, _NPAD)),
                     jnp.broadcast_to(batch_cls(1), (8, _NPAD)))

    def step(t, carry):
        acc_s, ax1, ay1, ax2, ay2, acls, aval = carry
        s = s_ref[...]
        m = jnp.max(s, axis=1, keepdims=True)
        eqm = s == m
        idxv = jnp.min(jnp.where(eqm, li, _NPAD), axis=1, keepdims=True)
        onehot = li == idxv

        def fetch(v):
            return jnp.sum(jnp.where(onehot, v, 0.0), axis=1, keepdims=True)

        bx1 = fetch(x1a)
        by1 = fetch(y1a)
        bx2 = fetch(x2a)
        by2 = fetch(y2a)
        bcl = fetch(cls8)
        valid = m > _SCORE_THR
        validf = valid.astype(f32)

        xx1 = jnp.maximum(bx1, x1a)
        yy1 = jnp.maximum(by1, y1a)
        xx2 = jnp.minimum(bx2, x2a)
        yy2 = jnp.minimum(by2, y2a)
        inter = jnp.maximum(xx2 - xx1, 0.0) * jnp.maximum(yy2 - yy1, 0.0)
        a_i = (bx2 - bx1) * (by2 - by1)
        iou = inter / (a_i + areas8 - inter + 1e-8)
        weight = jnp.exp(-0.5 * iou * iou / _SIGMA)
        ns = s * weight
        keep = (ns > _SCORE_THR) & jnp.logical_not(onehot)
        s_ref[...] = jnp.where(keep, ns, _NEG)

        colhot = (ki == t).astype(f32)
        ssel = jnp.where(valid, m, 0.0)
        acc_s = acc_s + colhot * (ssel * validf)
        ax1 = ax1 + colhot * (bx1 * validf)
        ay1 = ay1 + colhot * (by1 * validf)
        ax2 = ax2 + colhot * (bx2 * validf)
        ay2 = ay2 + colhot * (by2 * validf)
        acls = acls + colhot * (bcl * validf)
        aval = aval + colhot * validf
        return acc_s, ax1, ay1, ax2, ay2, acls, aval

    zeros8 = jnp.zeros((8, _LANE), f32)
    acc_s, ax1, ay1, ax2, ay2, acls, aval = jax.lax.fori_loop(
        0, _MAX_PER_CLASS, step,
        (zeros8, zeros8, zeros8, zeros8, zeros8, zeros8, zeros8))

    okey = (jax.lax.broadcasted_iota(jnp.int32, (4, _LANE), 0) * _LANE
            + jax.lax.broadcasted_iota(jnp.int32, (4, _LANE), 1))
    kl = jax.lax.broadcasted_iota(jnp.int32, (1, _LANE), 1)
    BIG = jnp.int32(1 << 20)

    for b in range(_B):
        r0, r1 = 4 * b, 4 * b + 4
        v4 = aval[r0:r1] > 0.0
        s4 = acc_s[r0:r1]
        x14 = ax1[r0:r1]
        y14 = ay1[r0:r1]
        x24 = ax2[r0:r1]
        y24 = ay2[r0:r1]
        c4 = acls[r0:r1]
        nv = jnp.sum(v4.astype(jnp.int32), axis=(0, 1), keepdims=True)

        def sel_sums(hot):
            def red(v):
                return jnp.sum(hot * v, axis=(0, 1), keepdims=True)
            return red(x14), red(y14), red(x24), red(y24), red(s4), red(c4)

        def cstep(k, carry):
            taken, ox1, oy1, ox2, oy2, osc, ocl = carry
            mask = v4 & (taken > 0.0)
            key = jnp.where(mask, okey, BIG)
            mk = jnp.min(key, axis=(0, 1), keepdims=True)
            sel = (okey == mk) & mask
            hot = sel.astype(jnp.float32)
            vx1, vy1, vx2, vy2, vsc, vcl = sel_sums(hot)
            khot = (kl == k).astype(jnp.float32)
            ox1 = ox1 + khot * vx1
            oy1 = oy1 + khot * vy1
            ox2 = ox2 + khot * vx2
            oy2 = oy2 + khot * vy2
            osc = osc + khot * vsc
            ocl = ocl + khot * vcl
            taken = jnp.where(sel, 0.0, taken)
            return taken, ox1, oy1, ox2, oy2, osc, ocl

        z1 = jnp.zeros((1, _LANE), jnp.float32)
        ones4 = jnp.ones((4, _LANE), jnp.float32)
        _, ox1, oy1, ox2, oy2, osc, ocl = jax.lax.fori_loop(
            0, _MAX_DET, cstep, (ones4, z1, z1, z1, z1, z1, z1))

        def tstep(k, carry):
            taken, px1, py1, px2, py2, psc = carry
            mask = taken > 0.0
            sc = jnp.where(mask, s4, -1.0)
            ms = jnp.max(sc, axis=(0, 1), keepdims=True)
            kk = jnp.where(sc == ms, okey, BIG)
            mkk = jnp.min(kk, axis=(0, 1), keepdims=True)
            sel = okey == mkk
            hot = sel.astype(jnp.float32)
            vx1, vy1, vx2, vy2, _, _ = sel_sums(hot)
            khot = (kl == k).astype(jnp.float32)
            px1 = px1 + khot * vx1
            py1 = py1 + khot * vy1
            px2 = px2 + khot * vx2
            py2 = py2 + khot * vy2
            psc = psc + khot * ms
            taken = jnp.where(sel, 0.0, taken)
            return taken, px1, py1, px2, py2, psc

        _, px1, py1, px2, py2, psc = jax.lax.fori_loop(
            0, _MAX_DET, tstep, (ones4, z1, z1, z1, z1, z1))

        def lane_val(vec, j):
            return jnp.sum(jnp.where(kl == j, vec, 0.0), axis=(0, 1),
                           keepdims=True)
        cc0 = lane_val(ocl, 0)
        cc1 = lane_val(ocl, 1)
        cc2 = lane_val(ocl, 2)
        cc3 = lane_val(ocl, 3)
        buggy = jnp.where(ocl == 0.0, cc0,
                          jnp.where(ocl == 1.0, cc1,
                                    jnp.where(ocl == 2.0, cc2, cc3)))

        use_keep = nv <= _MAX_DET
        fx1 = jnp.where(use_keep, ox1, px1)
        fy1 = jnp.where(use_keep, oy1, py1)
        fx2 = jnp.where(use_keep, ox2, px2)
        fy2 = jnp.where(use_keep, oy2, py2)
        fsc = jnp.where(use_keep, osc, psc)
        ckeep = jnp.where(kl < nv, ocl, -1.0)
        fcl = jnp.where(use_keep, ckeep, buggy)

        bx_ref[b] = jnp.concatenate([fx1, fy1, fx2, fy2], axis=0)
        sc_ref[b] = fsc
        cl_ref[b] = fcl.astype(jnp.int32)
        vd_ref[b] = jnp.broadcast_to(jnp.minimum(nv, _MAX_DET), (1, _LANE))


def kernel(predictions):
    p = predictions.astype(jnp.float32)
    box_t = jnp.transpose(p[:, :, :4], (0, 2, 1))
    box_t = jnp.pad(box_t, ((0, 0), (0, 0), (0, _NPAD - _N)))
    cls_t = jnp.transpose(p[:, :, 4:], (0, 2, 1)).reshape(8, _N)
    cls_t = jnp.pad(cls_t, ((0, 0), (0, _NPAD - _N)),
                    constant_values=-1e30)
    anc = jnp.asarray(_ANCHORS_NP.T, jnp.float32)
    anc = jnp.pad(anc, ((0, 0), (0, _NPAD - _N)))

    out_shape = [
        jax.ShapeDtypeStruct((_B, 1, _LANE), jnp.int32),
        jax.ShapeDtypeStruct((_B, 4, _LANE), jnp.float32),
        jax.ShapeDtypeStruct((_B, 1, _LANE), jnp.float32),
        jax.ShapeDtypeStruct((_B, 1, _LANE), jnp.int32),
    ]
    vd, bx, sc, cl = pl.pallas_call(
        _nms_body,
        out_shape=out_shape,
        scratch_shapes=[
            pltpu.VMEM((8, _NPAD), jnp.float32),
        ],
    )(box_t, cls_t, anc)

    valid_detections = vd[:, 0, 0]
    nmsed_boxes = jnp.transpose(bx, (0, 2, 1))[:, :_MAX_DET, :]
    nmsed_scores = sc[:, 0, :_MAX_DET]
    nmsed_classes = cl[:, 0, :_MAX_DET]
    return valid_detections, nmsed_boxes, nmsed_scores, nmsed_classes


# speedup vs baseline: 2.6759x; 2.2742x over previous
import numpy as np
import jax
import jax.numpy as jnp
from jax.experimental import pallas as pl
from jax.experimental.pallas import tpu as pltpu

_NUM_CLASSES = 4
_IMAGE_SHAPE = (256, 256)
_SCORE_THR = 0.05
_SIGMA = 0.05
_MAX_PER_CLASS = 100
_MAX_DET = 100

_B = 2
_LANE = 128
_NEG = -jnp.inf


def _gen_anchors(image_shape):
    aspect_ratios = [0.5, 1.0, 2.0]
    scales = [2.0 ** x for x in [0.0, 1.0 / 3.0, 2.0 / 3.0]]
    areas = [float(x) ** 2 for x in [32, 64, 128, 256, 512]]
    all_anchors = []
    for level, area in zip(range(3, 8), areas):
        stride = 2 ** level
        dims = []
        for ratio in aspect_ratios:
            h = np.sqrt(area / ratio)
            w = area / h
            for s in scales:
                dims.append([w * s, h * s])
        dims = np.asarray(dims, np.float32)
        fh = int(np.ceil(image_shape[0] / stride))
        fw = int(np.ceil(image_shape[1] / stride))
        cx = (np.arange(fw, dtype=np.float32) + 0.5) * stride
        cy = (np.arange(fh, dtype=np.float32) + 0.5) * stride
        cxg, cyg = np.meshgrid(cx, cy)
        centers = np.stack([cxg, cyg], axis=-1).reshape(-1, 1, 2)
        centers = np.tile(centers, (1, dims.shape[0], 1))
        d = np.tile(dims[None, :, :], (centers.shape[0], 1, 1))
        all_anchors.append(np.concatenate([centers, d], axis=-1).reshape(-1, 4))
    return np.concatenate(all_anchors, axis=0)


_ANCHORS_NP = _gen_anchors(_IMAGE_SHAPE)
_N = _ANCHORS_NP.shape[0]
_NPAD = ((_N + _LANE - 1) // _LANE) * _LANE


def _nms_body(box_ref, cls_ref, anc_ref,
              vd_ref, bx_ref, sc_ref, cl_ref, s_ref):
    f32 = jnp.float32
    rowid = jax.lax.broadcasted_iota(jnp.int32, (8, _NPAD), 0)
    li = jax.lax.broadcasted_iota(jnp.int32, (8, _NPAD), 1)
    ki = jax.lax.broadcasted_iota(jnp.int32, (8, _LANE), 1)

    acx = anc_ref[0:1, :]
    acy = anc_ref[1:2, :]
    aw = anc_ref[2:3, :]
    ah = anc_ref[3:4, :]

    coords = []
    for b in range(_B):
        tx = box_ref[b, 0:1, :]
        ty = box_ref[b, 1:2, :]
        tw = box_ref[b, 2:3, :]
        th = box_ref[b, 3:4, :]
        cx = tx * aw + acx
        cy = ty * ah + acy
        w = jnp.exp(tw) * aw
        h = jnp.exp(th) * ah
        x1 = cx - w / 2.0
        y1 = cy - h / 2.0
        x2 = cx + w / 2.0
        y2 = cy + h / 2.0
        ar = (x2 - x1) * (y2 - y1)
        coords.append((x1, y1, x2, y2, ar))

    def expand(v0, v1):
        return jnp.where(rowid < 4, v0, v1)

    x1a = expand(coords[0][0], coords[1][0])
    y1a = expand(coords[0][1], coords[1][1])
    x2a = expand(coords[0][2], coords[1][2])
    y2a = expand(coords[0][3], coords[1][3])
    areas8 = expand(coords[0][4], coords[1][4])

    scores0 = jax.nn.sigmoid(cls_ref[...])
    s_ref[...] = jnp.where(scores0 > _SCORE_THR, scores0, _NEG)

    def batch_cls(b):
        best = scores0[4 * b: 4 * b + 1, :]
        bidx = jnp.zeros_like(best)
        for c in range(1, 4):
            sc = scores0[4 * b + c: 4 * b + c + 1, :]
            gt = sc > best
            bidx = jnp.where(gt, f32(c), bidx)
            best = jnp.maximum(sc, best)
        return bidx
    cls8 = jnp.where(rowid < 4,
                     jnp.broadcast_to(batch_cls(0), (8, _NPAD)),
                     jnp.broadcast_to(batch_cls(1), (8, _NPAD)))

    def step(t, carry):
        acc_s, ax1, ay1, ax2, ay2, acls, aval = carry
        s = s_ref[...]
        m = jnp.max(s, axis=1, keepdims=True)
        eqm = s == m
        idxv = jnp.min(jnp.where(eqm, li, _NPAD), axis=1, keepdims=True)
        onehot = li == idxv

        def fetch(v):
            return jnp.sum(jnp.where(onehot, v, 0.0), axis=1, keepdims=True)

        bx1 = fetch(x1a)
        by1 = fetch(y1a)
        bx2 = fetch(x2a)
        by2 = fetch(y2a)
        bcl = fetch(cls8)
        valid = m > _SCORE_THR
        validf = valid.astype(f32)

        xx1 = jnp.maximum(bx1, x1a)
        yy1 = jnp.maximum(by1, y1a)
        xx2 = jnp.minimum(bx2, x2a)
        yy2 = jnp.minimum(by2, y2a)
        inter = jnp.maximum(xx2 - xx1, 0.0) * jnp.maximum(yy2 - yy1, 0.0)
        a_i = (bx2 - bx1) * (by2 - by1)
        iou = inter / (a_i + areas8 - inter + 1e-8)
        weight = jnp.exp(-0.5 * iou * iou / _SIGMA)
        ns = s * weight
        keep = (ns > _SCORE_THR) & jnp.logical_not(onehot)
        s_ref[...] = jnp.where(keep, ns, _NEG)

        colhot = (ki == t).astype(f32)
        ssel = jnp.where(valid, m, 0.0)
        acc_s = acc_s + colhot * (ssel * validf)
        ax1 = ax1 + colhot * (bx1 * validf)
        ay1 = ay1 + colhot * (by1 * validf)
        ax2 = ax2 + colhot * (bx2 * validf)
        ay2 = ay2 + colhot * (by2 * validf)
        acls = acls + colhot * (bcl * validf)
        aval = aval + colhot * validf
        return acc_s, ax1, ay1, ax2, ay2, acls, aval

    zeros8 = jnp.zeros((8, _LANE), f32)
    acc_s, ax1, ay1, ax2, ay2, acls, aval = jax.lax.fori_loop(
        0, _MAX_PER_CLASS, step,
        (zeros8, zeros8, zeros8, zeros8, zeros8, zeros8, zeros8))

    r_iota = jax.lax.broadcasted_iota(jnp.int32, (_LANE, _LANE), 0)
    c_iota = jax.lax.broadcasted_iota(jnp.int32, (_LANE, _LANE), 1)
    sut = (r_iota < c_iota).astype(f32)
    kf = c_iota.astype(f32)
    kl = jax.lax.broadcasted_iota(jnp.int32, (1, _LANE), 1)
    hiP = jax.lax.Precision.HIGHEST

    v8 = aval > 0.0
    pre = jnp.dot(aval, sut, preferred_element_type=f32)
    rt = jnp.sum(aval, axis=1, keepdims=True)
    offs_rows = []
    nvs = []
    for b in range(_B):
        acc0 = jnp.zeros((1, 1), f32)
        for r in range(4):
            offs_rows.append(acc0)
            acc0 = acc0 + rt[4 * b + r: 4 * b + r + 1, 0:1]
        nvs.append(acc0)
    offs = jnp.concatenate(offs_rows, axis=0)
    rank_c = jnp.where(v8, pre + offs, 999.0)

    s_top = jnp.where(v8, acc_s, -1.0)
    s_t = jnp.transpose(s_top)
    rank_rows = []
    for b in range(_B):
        for rj in range(4):
            srow = s_top[4 * b + rj: 4 * b + rj + 1, :]
            cnt_acc = None
            for ri in range(4):
                scol = s_t[:, 4 * b + ri: 4 * b + ri + 1]
                gt = scol > srow
                if ri == rj:
                    big = gt | ((scol == srow) & (r_iota < c_iota))
                elif ri < rj:
                    big = gt | (scol == srow)
                else:
                    big = gt
                cnt = jnp.sum(big.astype(f32), axis=0, keepdims=True)
                cnt_acc = cnt if cnt_acc is None else cnt_acc + cnt
            rank_rows.append(cnt_acc)
    rank_t = jnp.concatenate(rank_rows, axis=0)

    ranks_tr = jnp.transpose(jnp.concatenate([rank_c, rank_t], axis=0))

    for b in range(_B):
        keep6 = jnp.zeros((6, _LANE), f32)
        top6 = jnp.zeros((6, _LANE), f32)
        for j in range(4):
            r = 4 * b + j
            vals = jnp.concatenate(
                [ax1[r:r + 1], ay1[r:r + 1], ax2[r:r + 1], ay2[r:r + 1],
                 acc_s[r:r + 1], acls[r:r + 1]], axis=0)
            p_c = (ranks_tr[:, r:r + 1] == kf).astype(f32)
            p_t = (ranks_tr[:, 8 + r:8 + r + 1] == kf).astype(f32)
            keep6 = keep6 + jnp.dot(vals, p_c, precision=hiP,
                                    preferred_element_type=f32)
            top6 = top6 + jnp.dot(vals, p_t, precision=hiP,
                                  preferred_element_type=f32)
        ox1, oy1 = keep6[0:1], keep6[1:2]
        ox2, oy2 = keep6[2:3], keep6[3:4]
        osc, ocl = keep6[4:5], keep6[5:6]
        px1, py1 = top6[0:1], top6[1:2]
        px2, py2 = top6[2:3], top6[3:4]
        psc = top6[4:5]
        nv = nvs[b]

        def lane_val(vec, j):
            return jnp.sum(jnp.where(kl == j, vec, 0.0), axis=(0, 1),
                           keepdims=True)
        cc0 = lane_val(ocl, 0)
        cc1 = lane_val(ocl, 1)
        cc2 = lane_val(ocl, 2)
        cc3 = lane_val(ocl, 3)
        buggy = jnp.where(ocl == 0.0, cc0,
                          jnp.where(ocl == 1.0, cc1,
                                    jnp.where(ocl == 2.0, cc2, cc3)))

        use_keep = nv <= f32(_MAX_DET)
        fx1 = jnp.where(use_keep, ox1, px1)
        fy1 = jnp.where(use_keep, oy1, py1)
        fx2 = jnp.where(use_keep, ox2, px2)
        fy2 = jnp.where(use_keep, oy2, py2)
        fsc = jnp.where(use_keep, osc, psc)
        ckeep = jnp.where(kl < nv.astype(jnp.int32), ocl, -1.0)
        fcl = jnp.where(use_keep, ckeep, buggy)

        bx_ref[b] = jnp.concatenate([fx1, fy1, fx2, fy2], axis=0)
        sc_ref[b] = fsc
        cl_ref[b] = fcl.astype(jnp.int32)
        vd_ref[b] = jnp.broadcast_to(
            jnp.minimum(nv, f32(_MAX_DET)).astype(jnp.int32), (1, _LANE))


def kernel(predictions):
    p = predictions.astype(jnp.float32)
    box_t = jnp.transpose(p[:, :, :4], (0, 2, 1))
    box_t = jnp.pad(box_t, ((0, 0), (0, 0), (0, _NPAD - _N)))
    cls_t = jnp.transpose(p[:, :, 4:], (0, 2, 1)).reshape(8, _N)
    cls_t = jnp.pad(cls_t, ((0, 0), (0, _NPAD - _N)),
                    constant_values=-1e30)
    anc = jnp.asarray(_ANCHORS_NP.T, jnp.float32)
    anc = jnp.pad(anc, ((0, 0), (0, _NPAD - _N)))

    out_shape = [
        jax.ShapeDtypeStruct((_B, 1, _LANE), jnp.int32),
        jax.ShapeDtypeStruct((_B, 4, _LANE), jnp.float32),
        jax.ShapeDtypeStruct((_B, 1, _LANE), jnp.float32),
        jax.ShapeDtypeStruct((_B, 1, _LANE), jnp.int32),
    ]
    vd, bx, sc, cl = pl.pallas_call(
        _nms_body,
        out_shape=out_shape,
        scratch_shapes=[
            pltpu.VMEM((8, _NPAD), jnp.float32),
        ],
    )(box_t, cls_t, anc)

    valid_detections = vd[:, 0, 0]
    nmsed_boxes = jnp.transpose(bx, (0, 2, 1))[:, :_MAX_DET, :]
    nmsed_scores = sc[:, 0, :_MAX_DET]
    nmsed_classes = cl[:, 0, :_MAX_DET]
    return valid_detections, nmsed_boxes, nmsed_scores, nmsed_classes


# speedup vs baseline: 2.9226x; 1.0922x over previous
import numpy as np
import jax
import jax.numpy as jnp
from jax.experimental import pallas as pl
from jax.experimental.pallas import tpu as pltpu

_NUM_CLASSES = 4
_IMAGE_SHAPE = (256, 256)
_SCORE_THR = 0.05
_SIGMA = 0.05
_MAX_PER_CLASS = 100
_MAX_DET = 100

_B = 2
_LANE = 128
_NEG = -jnp.inf


def _gen_anchors(image_shape):
    aspect_ratios = [0.5, 1.0, 2.0]
    scales = [2.0 ** x for x in [0.0, 1.0 / 3.0, 2.0 / 3.0]]
    areas = [float(x) ** 2 for x in [32, 64, 128, 256, 512]]
    all_anchors = []
    for level, area in zip(range(3, 8), areas):
        stride = 2 ** level
        dims = []
        for ratio in aspect_ratios:
            h = np.sqrt(area / ratio)
            w = area / h
            for s in scales:
                dims.append([w * s, h * s])
        dims = np.asarray(dims, np.float32)
        fh = int(np.ceil(image_shape[0] / stride))
        fw = int(np.ceil(image_shape[1] / stride))
        cx = (np.arange(fw, dtype=np.float32) + 0.5) * stride
        cy = (np.arange(fh, dtype=np.float32) + 0.5) * stride
        cxg, cyg = np.meshgrid(cx, cy)
        centers = np.stack([cxg, cyg], axis=-1).reshape(-1, 1, 2)
        centers = np.tile(centers, (1, dims.shape[0], 1))
        d = np.tile(dims[None, :, :], (centers.shape[0], 1, 1))
        all_anchors.append(np.concatenate([centers, d], axis=-1).reshape(-1, 4))
    return np.concatenate(all_anchors, axis=0)


_ANCHORS_NP = _gen_anchors(_IMAGE_SHAPE)
_N = _ANCHORS_NP.shape[0]
_NPAD = ((_N + _LANE - 1) // _LANE) * _LANE


def _nms_body(box_ref, cls_ref, anc_ref,
              vd_ref, bx_ref, sc_ref, cl_ref, s_ref):
    f32 = jnp.float32
    rowid = jax.lax.broadcasted_iota(jnp.int32, (8, _NPAD), 0)
    li = jax.lax.broadcasted_iota(jnp.int32, (8, _NPAD), 1)
    ki = jax.lax.broadcasted_iota(jnp.int32, (8, _LANE), 1)

    acx = anc_ref[0:1, :]
    acy = anc_ref[1:2, :]
    aw = anc_ref[2:3, :]
    ah = anc_ref[3:4, :]

    coords = []
    for b in range(_B):
        tx = box_ref[b, 0:1, :]
        ty = box_ref[b, 1:2, :]
        tw = box_ref[b, 2:3, :]
        th = box_ref[b, 3:4, :]
        cx = tx * aw + acx
        cy = ty * ah + acy
        w = jnp.exp(tw) * aw
        h = jnp.exp(th) * ah
        x1 = cx - w / 2.0
        y1 = cy - h / 2.0
        x2 = cx + w / 2.0
        y2 = cy + h / 2.0
        ar = (x2 - x1) * (y2 - y1)
        coords.append((x1, y1, x2, y2, ar))

    def expand(v0, v1):
        return jnp.where(rowid < 4, v0, v1)

    x1a = expand(coords[0][0], coords[1][0])
    y1a = expand(coords[0][1], coords[1][1])
    x2a = expand(coords[0][2], coords[1][2])
    y2a = expand(coords[0][3], coords[1][3])
    areas8 = expand(coords[0][4], coords[1][4])

    scores0 = jax.nn.sigmoid(cls_ref[...])
    s_init = jnp.where(scores0 > _SCORE_THR, scores0, _NEG)
    s_ref[...] = s_init
    m_init = jnp.max(s_init, axis=1, keepdims=True)

    def batch_cls(b):
        best = scores0[4 * b: 4 * b + 1, :]
        bidx = jnp.zeros_like(best)
        for c in range(1, 4):
            sc = scores0[4 * b + c: 4 * b + c + 1, :]
            gt = sc > best
            bidx = jnp.where(gt, f32(c), bidx)
            best = jnp.maximum(sc, best)
        return bidx
    cls8 = jnp.where(rowid < 4,
                     jnp.broadcast_to(batch_cls(0), (8, _NPAD)),
                     jnp.broadcast_to(batch_cls(1), (8, _NPAD)))

    def step(t, carry):
        m, acc_s, ax1, ay1, ax2, ay2, acls, aval = carry
        s = s_ref[...]
        eqm = s == m
        idxv = jnp.min(jnp.where(eqm, li, _NPAD), axis=1, keepdims=True)
        onehot = li == idxv

        def fetch(v):
            return jnp.sum(jnp.where(onehot, v, 0.0), axis=1, keepdims=True)

        bx1 = fetch(x1a)
        by1 = fetch(y1a)
        bx2 = fetch(x2a)
        by2 = fetch(y2a)
        bcl = fetch(cls8)
        valid = m > _SCORE_THR
        validf = valid.astype(f32)

        xx1 = jnp.maximum(bx1, x1a)
        yy1 = jnp.maximum(by1, y1a)
        xx2 = jnp.minimum(bx2, x2a)
        yy2 = jnp.minimum(by2, y2a)
        inter = jnp.maximum(xx2 - xx1, 0.0) * jnp.maximum(yy2 - yy1, 0.0)
        a_i = (bx2 - bx1) * (by2 - by1)
        iou = inter / (a_i + areas8 - inter + 1e-8)
        weight = jnp.exp(-0.5 * iou * iou / _SIGMA)
        ns = s * weight
        keep = (ns > _SCORE_THR) & jnp.logical_not(onehot)
        s_new = jnp.where(keep, ns, _NEG)
        s_ref[...] = s_new
        m_next = jnp.max(s_new, axis=1, keepdims=True)

        colhot = (ki == t).astype(f32)
        ssel = jnp.where(valid, m, 0.0)
        acc_s = acc_s + colhot * (ssel * validf)
        ax1 = ax1 + colhot * (bx1 * validf)
        ay1 = ay1 + colhot * (by1 * validf)
        ax2 = ax2 + colhot * (bx2 * validf)
        ay2 = ay2 + colhot * (by2 * validf)
        acls = acls + colhot * (bcl * validf)
        aval = aval + colhot * validf
        return m_next, acc_s, ax1, ay1, ax2, ay2, acls, aval

    zeros8 = jnp.zeros((8, _LANE), f32)
    _, acc_s, ax1, ay1, ax2, ay2, acls, aval = jax.lax.fori_loop(
        0, _MAX_PER_CLASS, step,
        (m_init, zeros8, zeros8, zeros8, zeros8, zeros8, zeros8, zeros8),
        unroll=2)

    r_iota = jax.lax.broadcasted_iota(jnp.int32, (_LANE, _LANE), 0)
    c_iota = jax.lax.broadcasted_iota(jnp.int32, (_LANE, _LANE), 1)
    sut = (r_iota < c_iota).astype(f32)
    kf = c_iota.astype(f32)
    kl = jax.lax.broadcasted_iota(jnp.int32, (1, _LANE), 1)
    hiP = jax.lax.Precision.HIGHEST

    v8 = aval > 0.0
    pre = jnp.dot(aval, sut, preferred_element_type=f32)
    rt = jnp.sum(aval, axis=1, keepdims=True)
    offs_rows = []
    nvs = []
    for b in range(_B):
        acc0 = jnp.zeros((1, 1), f32)
        for r in range(4):
            offs_rows.append(acc0)
            acc0 = acc0 + rt[4 * b + r: 4 * b + r + 1, 0:1]
        nvs.append(acc0)
    offs = jnp.concatenate(offs_rows, axis=0)
    rank_c = jnp.where(v8, pre + offs, 999.0)

    s_top = jnp.where(v8, acc_s, -1.0)
    s_t = jnp.transpose(s_top)
    rank_rows = []
    for b in range(_B):
        for rj in range(4):
            srow = s_top[4 * b + rj: 4 * b + rj + 1, :]
            cnt_acc = None
            for ri in range(4):
                scol = s_t[:, 4 * b + ri: 4 * b + ri + 1]
                gt = scol > srow
                if ri == rj:
                    big = gt | ((scol == srow) & (r_iota < c_iota))
                elif ri < rj:
                    big = gt | (scol == srow)
                else:
                    big = gt
                cnt = jnp.sum(big.astype(f32), axis=0, keepdims=True)
                cnt_acc = cnt if cnt_acc is None else cnt_acc + cnt
            rank_rows.append(cnt_acc)
    rank_t = jnp.concatenate(rank_rows, axis=0)

    ranks_tr = jnp.transpose(jnp.concatenate([rank_c, rank_t], axis=0))

    for b in range(_B):
        keep6 = jnp.zeros((6, _LANE), f32)
        top6 = jnp.zeros((6, _LANE), f32)
        for j in range(4):
            r = 4 * b + j
            vals = jnp.concatenate(
                [ax1[r:r + 1], ay1[r:r + 1], ax2[r:r + 1], ay2[r:r + 1],
                 acc_s[r:r + 1], acls[r:r + 1]], axis=0)
            p_c = (ranks_tr[:, r:r + 1] == kf).astype(f32)
            p_t = (ranks_tr[:, 8 + r:8 + r + 1] == kf).astype(f32)
            keep6 = keep6 + jnp.dot(vals, p_c, precision=hiP,
                                    preferred_element_type=f32)
            top6 = top6 + jnp.dot(vals, p_t, precision=hiP,
                                  preferred_element_type=f32)
        ox1, oy1 = keep6[0:1], keep6[1:2]
        ox2, oy2 = keep6[2:3], keep6[3:4]
        osc, ocl = keep6[4:5], keep6[5:6]
        px1, py1 = top6[0:1], top6[1:2]
        px2, py2 = top6[2:3], top6[3:4]
        psc = top6[4:5]
        nv = nvs[b]

        def lane_val(vec, j):
            return jnp.sum(jnp.where(kl == j, vec, 0.0), axis=(0, 1),
                           keepdims=True)
        cc0 = lane_val(ocl, 0)
        cc1 = lane_val(ocl, 1)
        cc2 = lane_val(ocl, 2)
        cc3 = lane_val(ocl, 3)
        buggy = jnp.where(ocl == 0.0, cc0,
                          jnp.where(ocl == 1.0, cc1,
                                    jnp.where(ocl == 2.0, cc2, cc3)))

        use_keep = nv <= f32(_MAX_DET)
        fx1 = jnp.where(use_keep, ox1, px1)
        fy1 = jnp.where(use_keep, oy1, py1)
        fx2 = jnp.where(use_keep, ox2, px2)
        fy2 = jnp.where(use_keep, oy2, py2)
        fsc = jnp.where(use_keep, osc, psc)
        ckeep = jnp.where(kl < nv.astype(jnp.int32), ocl, -1.0)
        fcl = jnp.where(use_keep, ckeep, buggy)

        bx_ref[b] = jnp.concatenate([fx1, fy1, fx2, fy2], axis=0)
        sc_ref[b] = fsc
        cl_ref[b] = fcl.astype(jnp.int32)
        vd_ref[b] = jnp.broadcast_to(
            jnp.minimum(nv, f32(_MAX_DET)).astype(jnp.int32), (1, _LANE))


def kernel(predictions):
    p = predictions.astype(jnp.float32)
    box_t = jnp.transpose(p[:, :, :4], (0, 2, 1))
    box_t = jnp.pad(box_t, ((0, 0), (0, 0), (0, _NPAD - _N)))
    cls_t = jnp.transpose(p[:, :, 4:], (0, 2, 1)).reshape(8, _N)
    cls_t = jnp.pad(cls_t, ((0, 0), (0, _NPAD - _N)),
                    constant_values=-1e30)
    anc = jnp.asarray(_ANCHORS_NP.T, jnp.float32)
    anc = jnp.pad(anc, ((0, 0), (0, _NPAD - _N)))

    out_shape = [
        jax.ShapeDtypeStruct((_B, 1, _LANE), jnp.int32),
        jax.ShapeDtypeStruct((_B, 4, _LANE), jnp.float32),
        jax.ShapeDtypeStruct((_B, 1, _LANE), jnp.float32),
        jax.ShapeDtypeStruct((_B, 1, _LANE), jnp.int32),
    ]
    vd, bx, sc, cl = pl.pallas_call(
        _nms_body,
        out_shape=out_shape,
        scratch_shapes=[
            pltpu.VMEM((8, _NPAD), jnp.float32),
        ],
    )(box_t, cls_t, anc)

    valid_detections = vd[:, 0, 0]
    nmsed_boxes = jnp.transpose(bx, (0, 2, 1))[:, :_MAX_DET, :]
    nmsed_scores = sc[:, 0, :_MAX_DET]
    nmsed_classes = cl[:, 0, :_MAX_DET]
    return valid_detections, nmsed_boxes, nmsed_scores, nmsed_classes


# speedup vs baseline: 2.9463x; 1.0081x over previous
import numpy as np
import jax
import jax.numpy as jnp
from jax.experimental import pallas as pl
from jax.experimental.pallas import tpu as pltpu

_NUM_CLASSES = 4
_IMAGE_SHAPE = (256, 256)
_SCORE_THR = 0.05
_SIGMA = 0.05
_MAX_PER_CLASS = 100
_MAX_DET = 100

_B = 2
_LANE = 128
_NEG = -jnp.inf


def _gen_anchors(image_shape):
    aspect_ratios = [0.5, 1.0, 2.0]
    scales = [2.0 ** x for x in [0.0, 1.0 / 3.0, 2.0 / 3.0]]
    areas = [float(x) ** 2 for x in [32, 64, 128, 256, 512]]
    all_anchors = []
    for level, area in zip(range(3, 8), areas):
        stride = 2 ** level
        dims = []
        for ratio in aspect_ratios:
            h = np.sqrt(area / ratio)
            w = area / h
            for s in scales:
                dims.append([w * s, h * s])
        dims = np.asarray(dims, np.float32)
        fh = int(np.ceil(image_shape[0] / stride))
        fw = int(np.ceil(image_shape[1] / stride))
        cx = (np.arange(fw, dtype=np.float32) + 0.5) * stride
        cy = (np.arange(fh, dtype=np.float32) + 0.5) * stride
        cxg, cyg = np.meshgrid(cx, cy)
        centers = np.stack([cxg, cyg], axis=-1).reshape(-1, 1, 2)
        centers = np.tile(centers, (1, dims.shape[0], 1))
        d = np.tile(dims[None, :, :], (centers.shape[0], 1, 1))
        all_anchors.append(np.concatenate([centers, d], axis=-1).reshape(-1, 4))
    return np.concatenate(all_anchors, axis=0)


_ANCHORS_NP = _gen_anchors(_IMAGE_SHAPE)
_N = _ANCHORS_NP.shape[0]
_NPAD = ((_N + _LANE - 1) // _LANE) * _LANE


def _nms_body(box_ref, cls_ref, anc_ref,
              vd_ref, bx_ref, sc_ref, cl_ref, s_ref):
    f32 = jnp.float32
    rowid = jax.lax.broadcasted_iota(jnp.int32, (8, _NPAD), 0)
    li = jax.lax.broadcasted_iota(jnp.int32, (8, _NPAD), 1)
    ki = jax.lax.broadcasted_iota(jnp.int32, (8, _LANE), 1)

    acx = anc_ref[0:1, :]
    acy = anc_ref[1:2, :]
    aw = anc_ref[2:3, :]
    ah = anc_ref[3:4, :]

    coords = []
    for b in range(_B):
        tx = box_ref[b, 0:1, :]
        ty = box_ref[b, 1:2, :]
        tw = box_ref[b, 2:3, :]
        th = box_ref[b, 3:4, :]
        cx = tx * aw + acx
        cy = ty * ah + acy
        w = jnp.exp(tw) * aw
        h = jnp.exp(th) * ah
        x1 = cx - w / 2.0
        y1 = cy - h / 2.0
        x2 = cx + w / 2.0
        y2 = cy + h / 2.0
        ar = (x2 - x1) * (y2 - y1)
        coords.append((x1, y1, x2, y2, ar))

    def expand(v0, v1):
        return jnp.where(rowid < 4, v0, v1)

    x1a = expand(coords[0][0], coords[1][0])
    y1a = expand(coords[0][1], coords[1][1])
    x2a = expand(coords[0][2], coords[1][2])
    y2a = expand(coords[0][3], coords[1][3])
    areas8 = expand(coords[0][4], coords[1][4])

    scores0 = jax.nn.sigmoid(cls_ref[...])
    s_init = jnp.where(scores0 > _SCORE_THR, scores0, _NEG)
    s_ref[...] = s_init
    m_init = jnp.max(s_init, axis=1, keepdims=True)

    def batch_cls(b):
        best = scores0[4 * b: 4 * b + 1, :]
        bidx = jnp.zeros_like(best, dtype=jnp.int32)
        for c in range(1, 4):
            sc = scores0[4 * b + c: 4 * b + c + 1, :]
            bidx = jnp.where(sc > best, jnp.int32(c), bidx)
            best = jnp.maximum(sc, best)
        return bidx
    cls8i = jnp.where(rowid < 4,
                      jnp.broadcast_to(batch_cls(0), (8, _NPAD)),
                      jnp.broadcast_to(batch_cls(1), (8, _NPAD)))
    li4c = li * 4 + cls8i
    _BIGKEY = jnp.int32(_NPAD * 4)

    def step(t, carry):
        m, acc_s, ax1, ay1, ax2, ay2, acls, aval = carry
        s = s_ref[...]
        eqm = s == m
        idx2 = jnp.min(jnp.where(eqm, li4c, _BIGKEY), axis=1, keepdims=True)
        onehot = li4c == idx2
        bcl = (idx2 & 3).astype(f32)

        def fetch(v):
            return jnp.sum(jnp.where(onehot, v, 0.0), axis=1, keepdims=True)

        bx1 = fetch(x1a)
        by1 = fetch(y1a)
        bx2 = fetch(x2a)
        by2 = fetch(y2a)
        valid = m > _SCORE_THR
        validf = valid.astype(f32)

        xx1 = jnp.maximum(bx1, x1a)
        yy1 = jnp.maximum(by1, y1a)
        xx2 = jnp.minimum(bx2, x2a)
        yy2 = jnp.minimum(by2, y2a)
        inter = jnp.maximum(xx2 - xx1, 0.0) * jnp.maximum(yy2 - yy1, 0.0)
        a_i = (bx2 - bx1) * (by2 - by1)
        iou = inter / (a_i + areas8 - inter + 1e-8)
        weight = jnp.exp(-0.5 * iou * iou / _SIGMA)
        ns = s * weight
        keep = (ns > _SCORE_THR) & jnp.logical_not(onehot)
        s_new = jnp.where(keep, ns, _NEG)
        s_ref[...] = s_new
        m_next = jnp.max(s_new, axis=1, keepdims=True)

        colhot = (ki == t).astype(f32)
        ssel = jnp.where(valid, m, 0.0)
        acc_s = acc_s + colhot * (ssel * validf)
        ax1 = ax1 + colhot * (bx1 * validf)
        ay1 = ay1 + colhot * (by1 * validf)
        ax2 = ax2 + colhot * (bx2 * validf)
        ay2 = ay2 + colhot * (by2 * validf)
        acls = acls + colhot * (bcl * validf)
        aval = aval + colhot * validf
        return m_next, acc_s, ax1, ay1, ax2, ay2, acls, aval

    zeros8 = jnp.zeros((8, _LANE), f32)
    _, acc_s, ax1, ay1, ax2, ay2, acls, aval = jax.lax.fori_loop(
        0, _MAX_PER_CLASS, step,
        (m_init, zeros8, zeros8, zeros8, zeros8, zeros8, zeros8, zeros8),
        unroll=2)

    r_iota = jax.lax.broadcasted_iota(jnp.int32, (_LANE, _LANE), 0)
    c_iota = jax.lax.broadcasted_iota(jnp.int32, (_LANE, _LANE), 1)
    sut = (r_iota < c_iota).astype(f32)
    kf = c_iota.astype(f32)
    kl = jax.lax.broadcasted_iota(jnp.int32, (1, _LANE), 1)
    hiP = jax.lax.Precision.HIGHEST

    v8 = aval > 0.0
    pre = jnp.dot(aval, sut, preferred_element_type=f32)
    rt = jnp.sum(aval, axis=1, keepdims=True)
    offs_rows = []
    nvs = []
    for b in range(_B):
        acc0 = jnp.zeros((1, 1), f32)
        for r in range(4):
            offs_rows.append(acc0)
            acc0 = acc0 + rt[4 * b + r: 4 * b + r + 1, 0:1]
        nvs.append(acc0)
    offs = jnp.concatenate(offs_rows, axis=0)
    rank_c = jnp.where(v8, pre + offs, 999.0)

    s_top = jnp.where(v8, acc_s, -1.0)
    s_t = jnp.transpose(s_top)
    rank_rows = []
    for b in range(_B):
        for rj in range(4):
            srow = s_top[4 * b + rj: 4 * b + rj + 1, :]
            cnt_acc = None
            for ri in range(4):
                scol = s_t[:, 4 * b + ri: 4 * b + ri + 1]
                gt = scol > srow
                if ri == rj:
                    big = gt | ((scol == srow) & (r_iota < c_iota))
                elif ri < rj:
                    big = gt | (scol == srow)
                else:
                    big = gt
                cnt = jnp.sum(big.astype(f32), axis=0, keepdims=True)
                cnt_acc = cnt if cnt_acc is None else cnt_acc + cnt
            rank_rows.append(cnt_acc)
    rank_t = jnp.concatenate(rank_rows, axis=0)

    ranks_tr = jnp.transpose(jnp.concatenate([rank_c, rank_t], axis=0))

    for b in range(_B):
        keep6 = jnp.zeros((6, _LANE), f32)
        top6 = jnp.zeros((6, _LANE), f32)
        for j in range(4):
            r = 4 * b + j
            vals = jnp.concatenate(
                [ax1[r:r + 1], ay1[r:r + 1], ax2[r:r + 1], ay2[r:r + 1],
                 acc_s[r:r + 1], acls[r:r + 1]], axis=0)
            p_c = (ranks_tr[:, r:r + 1] == kf).astype(f32)
            p_t = (ranks_tr[:, 8 + r:8 + r + 1] == kf).astype(f32)
            keep6 = keep6 + jnp.dot(vals, p_c, precision=hiP,
                                    preferred_element_type=f32)
            top6 = top6 + jnp.dot(vals, p_t, precision=hiP,
                                  preferred_element_type=f32)
        ox1, oy1 = keep6[0:1], keep6[1:2]
        ox2, oy2 = keep6[2:3], keep6[3:4]
        osc, ocl = keep6[4:5], keep6[5:6]
        px1, py1 = top6[0:1], top6[1:2]
        px2, py2 = top6[2:3], top6[3:4]
        psc = top6[4:5]
        nv = nvs[b]

        def lane_val(vec, j):
            return jnp.sum(jnp.where(kl == j, vec, 0.0), axis=(0, 1),
                           keepdims=True)
        cc0 = lane_val(ocl, 0)
        cc1 = lane_val(ocl, 1)
        cc2 = lane_val(ocl, 2)
        cc3 = lane_val(ocl, 3)
        buggy = jnp.where(ocl == 0.0, cc0,
                          jnp.where(ocl == 1.0, cc1,
                                    jnp.where(ocl == 2.0, cc2, cc3)))

        use_keep = nv <= f32(_MAX_DET)
        fx1 = jnp.where(use_keep, ox1, px1)
        fy1 = jnp.where(use_keep, oy1, py1)
        fx2 = jnp.where(use_keep, ox2, px2)
        fy2 = jnp.where(use_keep, oy2, py2)
        fsc = jnp.where(use_keep, osc, psc)
        ckeep = jnp.where(kl < nv.astype(jnp.int32), ocl, -1.0)
        fcl = jnp.where(use_keep, ckeep, buggy)

        bx_ref[b] = jnp.concatenate([fx1, fy1, fx2, fy2], axis=0)
        sc_ref[b] = fsc
        cl_ref[b] = fcl.astype(jnp.int32)
        vd_ref[b] = jnp.broadcast_to(
            jnp.minimum(nv, f32(_MAX_DET)).astype(jnp.int32), (1, _LANE))


def kernel(predictions):
    p = predictions.astype(jnp.float32)
    box_t = jnp.transpose(p[:, :, :4], (0, 2, 1))
    box_t = jnp.pad(box_t, ((0, 0), (0, 0), (0, _NPAD - _N)))
    cls_t = jnp.transpose(p[:, :, 4:], (0, 2, 1)).reshape(8, _N)
    cls_t = jnp.pad(cls_t, ((0, 0), (0, _NPAD - _N)),
                    constant_values=-1e30)
    anc = jnp.asarray(_ANCHORS_NP.T, jnp.float32)
    anc = jnp.pad(anc, ((0, 0), (0, _NPAD - _N)))

    out_shape = [
        jax.ShapeDtypeStruct((_B, 1, _LANE), jnp.int32),
        jax.ShapeDtypeStruct((_B, 4, _LANE), jnp.float32),
        jax.ShapeDtypeStruct((_B, 1, _LANE), jnp.float32),
        jax.ShapeDtypeStruct((_B, 1, _LANE), jnp.int32),
    ]
    vd, bx, sc, cl = pl.pallas_call(
        _nms_body,
        out_shape=out_shape,
        scratch_shapes=[
            pltpu.VMEM((8, _NPAD), jnp.float32),
        ],
    )(box_t, cls_t, anc)

    valid_detections = vd[:, 0, 0]
    nmsed_boxes = jnp.transpose(bx, (0, 2, 1))[:, :_MAX_DET, :]
    nmsed_scores = sc[:, 0, :_MAX_DET]
    nmsed_classes = cl[:, 0, :_MAX_DET]
    return valid_detections, nmsed_boxes, nmsed_scores, nmsed_classes


# speedup vs baseline: 3.0545x; 1.0367x over previous
import numpy as np
import jax
import jax.numpy as jnp
from jax.experimental import pallas as pl
from jax.experimental.pallas import tpu as pltpu

_NUM_CLASSES = 4
_IMAGE_SHAPE = (256, 256)
_SCORE_THR = 0.05
_SIGMA = 0.05
_MAX_PER_CLASS = 100
_MAX_DET = 100

_B = 2
_LANE = 128
_NEG = -jnp.inf


def _gen_anchors(image_shape):
    aspect_ratios = [0.5, 1.0, 2.0]
    scales = [2.0 ** x for x in [0.0, 1.0 / 3.0, 2.0 / 3.0]]
    areas = [float(x) ** 2 for x in [32, 64, 128, 256, 512]]
    all_anchors = []
    for level, area in zip(range(3, 8), areas):
        stride = 2 ** level
        dims = []
        for ratio in aspect_ratios:
            h = np.sqrt(area / ratio)
            w = area / h
            for s in scales:
                dims.append([w * s, h * s])
        dims = np.asarray(dims, np.float32)
        fh = int(np.ceil(image_shape[0] / stride))
        fw = int(np.ceil(image_shape[1] / stride))
        cx = (np.arange(fw, dtype=np.float32) + 0.5) * stride
        cy = (np.arange(fh, dtype=np.float32) + 0.5) * stride
        cxg, cyg = np.meshgrid(cx, cy)
        centers = np.stack([cxg, cyg], axis=-1).reshape(-1, 1, 2)
        centers = np.tile(centers, (1, dims.shape[0], 1))
        d = np.tile(dims[None, :, :], (centers.shape[0], 1, 1))
        all_anchors.append(np.concatenate([centers, d], axis=-1).reshape(-1, 4))
    return np.concatenate(all_anchors, axis=0)


_ANCHORS_NP = _gen_anchors(_IMAGE_SHAPE)
_N = _ANCHORS_NP.shape[0]
_NPAD = ((_N + _LANE - 1) // _LANE) * _LANE


def _nms_body(box_ref, cls_ref, anc_ref,
              vd_ref, bx_ref, sc_ref, cl_ref, s_ref):
    f32 = jnp.float32
    rowid = jax.lax.broadcasted_iota(jnp.int32, (8, _NPAD), 0)
    li = jax.lax.broadcasted_iota(jnp.int32, (8, _NPAD), 1)
    ki = jax.lax.broadcasted_iota(jnp.int32, (8, _LANE), 1)

    acx = anc_ref[0:1, :]
    acy = anc_ref[1:2, :]
    aw = anc_ref[2:3, :]
    ah = anc_ref[3:4, :]

    coords = []
    for b in range(_B):
        tx = box_ref[b, 0:1, :]
        ty = box_ref[b, 1:2, :]
        tw = box_ref[b, 2:3, :]
        th = box_ref[b, 3:4, :]
        cx = tx * aw + acx
        cy = ty * ah + acy
        w = jnp.exp(tw) * aw
        h = jnp.exp(th) * ah
        x1 = cx - w / 2.0
        y1 = cy - h / 2.0
        x2 = cx + w / 2.0
        y2 = cy + h / 2.0
        ar = (x2 - x1) * (y2 - y1)
        coords.append((x1, y1, x2, y2, ar))

    def expand(v0, v1):
        return jnp.where(rowid < 4, v0, v1)

    x1a = expand(coords[0][0], coords[1][0])
    y1a = expand(coords[0][1], coords[1][1])
    x2a = expand(coords[0][2], coords[1][2])
    y2a = expand(coords[0][3], coords[1][3])
    areas8 = expand(coords[0][4], coords[1][4])

    scores0 = jax.nn.sigmoid(cls_ref[...])
    s_init = jnp.where(scores0 > _SCORE_THR, scores0, _NEG)
    s_ref[...] = s_init
    m_init = jnp.max(s_init, axis=1, keepdims=True)

    def batch_cls(b):
        best = scores0[4 * b: 4 * b + 1, :]
        bidx = jnp.zeros_like(best, dtype=jnp.int32)
        for c in range(1, 4):
            sc = scores0[4 * b + c: 4 * b + c + 1, :]
            bidx = jnp.where(sc > best, jnp.int32(c), bidx)
            best = jnp.maximum(sc, best)
        return bidx
    cls8i = jnp.where(rowid < 4,
                      jnp.broadcast_to(batch_cls(0), (8, _NPAD)),
                      jnp.broadcast_to(batch_cls(1), (8, _NPAD)))
    li4c = li * 4 + cls8i
    _BIGKEY = jnp.int32(_NPAD * 4)

    def step(t, carry):
        m, acc_s, ax1, ay1, ax2, ay2, acls, aval = carry
        s = s_ref[...]
        eqm = s == m
        idx2 = jnp.min(jnp.where(eqm, li4c, _BIGKEY), axis=1, keepdims=True)
        onehot = li4c == idx2
        bcl = (idx2 & 3).astype(f32)

        def fetch(v):
            return jnp.sum(jnp.where(onehot, v, 0.0), axis=1, keepdims=True)

        bx1 = fetch(x1a)
        by1 = fetch(y1a)
        bx2 = fetch(x2a)
        by2 = fetch(y2a)
        valid = m > _SCORE_THR
        validf = valid.astype(f32)

        xx1 = jnp.maximum(bx1, x1a)
        yy1 = jnp.maximum(by1, y1a)
        xx2 = jnp.minimum(bx2, x2a)
        yy2 = jnp.minimum(by2, y2a)
        inter = jnp.maximum(xx2 - xx1, 0.0) * jnp.maximum(yy2 - yy1, 0.0)
        a_i = (bx2 - bx1) * (by2 - by1)
        iou = inter / (a_i + areas8 - inter + 1e-8)
        weight = jnp.exp(-0.5 * iou * iou / _SIGMA)
        ns = s * weight
        keep = (ns > _SCORE_THR) & jnp.logical_not(onehot)
        s_new = jnp.where(keep, ns, _NEG)
        s_ref[...] = s_new
        m_next = jnp.max(s_new, axis=1, keepdims=True)

        colhot = (ki == t).astype(f32)
        ssel = jnp.where(valid, m, 0.0)
        acc_s = acc_s + colhot * (ssel * validf)
        ax1 = ax1 + colhot * (bx1 * validf)
        ay1 = ay1 + colhot * (by1 * validf)
        ax2 = ax2 + colhot * (bx2 * validf)
        ay2 = ay2 + colhot * (by2 * validf)
        acls = acls + colhot * (bcl * validf)
        aval = aval + colhot * validf
        return m_next, acc_s, ax1, ay1, ax2, ay2, acls, aval

    zeros8 = jnp.zeros((8, _LANE), f32)
    _, acc_s, ax1, ay1, ax2, ay2, acls, aval = jax.lax.fori_loop(
        0, _MAX_PER_CLASS, step,
        (m_init, zeros8, zeros8, zeros8, zeros8, zeros8, zeros8, zeros8),
        unroll=4)

    r_iota = jax.lax.broadcasted_iota(jnp.int32, (_LANE, _LANE), 0)
    c_iota = jax.lax.broadcasted_iota(jnp.int32, (_LANE, _LANE), 1)
    sut = (r_iota < c_iota).astype(f32)
    kf = c_iota.astype(f32)
    kl = jax.lax.broadcasted_iota(jnp.int32, (1, _LANE), 1)
    hiP = jax.lax.Precision.HIGHEST

    v8 = aval > 0.0
    pre = jnp.dot(aval, sut, preferred_element_type=f32)
    rt = jnp.sum(aval, axis=1, keepdims=True)
    offs_rows = []
    nvs = []
    for b in range(_B):
        acc0 = jnp.zeros((1, 1), f32)
        for r in range(4):
            offs_rows.append(acc0)
            acc0 = acc0 + rt[4 * b + r: 4 * b + r + 1, 0:1]
        nvs.append(acc0)
    offs = jnp.concatenate(offs_rows, axis=0)
    rank_c = jnp.where(v8, pre + offs, 999.0)

    s_top = jnp.where(v8, acc_s, -1.0)
    s_t = jnp.transpose(s_top)
    rank_rows = []
    for b in range(_B):
        for rj in range(4):
            srow = s_top[4 * b + rj: 4 * b + rj + 1, :]
            cnt_acc = None
            for ri in range(4):
                scol = s_t[:, 4 * b + ri: 4 * b + ri + 1]
                gt = scol > srow
                if ri == rj:
                    big = gt | ((scol == srow) & (r_iota < c_iota))
                elif ri < rj:
                    big = gt | (scol == srow)
                else:
                    big = gt
                cnt = jnp.sum(big.astype(f32), axis=0, keepdims=True)
                cnt_acc = cnt if cnt_acc is None else cnt_acc + cnt
            rank_rows.append(cnt_acc)
    rank_t = jnp.concatenate(rank_rows, axis=0)

    ranks_tr = jnp.transpose(jnp.concatenate([rank_c, rank_t], axis=0))

    for b in range(_B):
        keep6 = jnp.zeros((6, _LANE), f32)
        top6 = jnp.zeros((6, _LANE), f32)
        for j in range(4):
            r = 4 * b + j
            vals = jnp.concatenate(
                [ax1[r:r + 1], ay1[r:r + 1], ax2[r:r + 1], ay2[r:r + 1],
                 acc_s[r:r + 1], acls[r:r + 1]], axis=0)
            p_c = (ranks_tr[:, r:r + 1] == kf).astype(f32)
            p_t = (ranks_tr[:, 8 + r:8 + r + 1] == kf).astype(f32)
            keep6 = keep6 + jnp.dot(vals, p_c, precision=hiP,
                                    preferred_element_type=f32)
            top6 = top6 + jnp.dot(vals, p_t, precision=hiP,
                                  preferred_element_type=f32)
        ox1, oy1 = keep6[0:1], keep6[1:2]
        ox2, oy2 = keep6[2:3], keep6[3:4]
        osc, ocl = keep6[4:5], keep6[5:6]
        px1, py1 = top6[0:1], top6[1:2]
        px2, py2 = top6[2:3], top6[3:4]
        psc = top6[4:5]
        nv = nvs[b]

        def lane_val(vec, j):
            return jnp.sum(jnp.where(kl == j, vec, 0.0), axis=(0, 1),
                           keepdims=True)
        cc0 = lane_val(ocl, 0)
        cc1 = lane_val(ocl, 1)
        cc2 = lane_val(ocl, 2)
        cc3 = lane_val(ocl, 3)
        buggy = jnp.where(ocl == 0.0, cc0,
                          jnp.where(ocl == 1.0, cc1,
                                    jnp.where(ocl == 2.0, cc2, cc3)))

        use_keep = nv <= f32(_MAX_DET)
        fx1 = jnp.where(use_keep, ox1, px1)
        fy1 = jnp.where(use_keep, oy1, py1)
        fx2 = jnp.where(use_keep, ox2, px2)
        fy2 = jnp.where(use_keep, oy2, py2)
        fsc = jnp.where(use_keep, osc, psc)
        ckeep = jnp.where(kl < nv.astype(jnp.int32), ocl, -1.0)
        fcl = jnp.where(use_keep, ckeep, buggy)

        bx_ref[b] = jnp.concatenate([fx1, fy1, fx2, fy2], axis=0)
        sc_ref[b] = fsc
        cl_ref[b] = fcl.astype(jnp.int32)
        vd_ref[b] = jnp.broadcast_to(
            jnp.minimum(nv, f32(_MAX_DET)).astype(jnp.int32), (1, _LANE))


def kernel(predictions):
    p = predictions.astype(jnp.float32)
    box_t = jnp.transpose(p[:, :, :4], (0, 2, 1))
    box_t = jnp.pad(box_t, ((0, 0), (0, 0), (0, _NPAD - _N)))
    cls_t = jnp.transpose(p[:, :, 4:], (0, 2, 1)).reshape(8, _N)
    cls_t = jnp.pad(cls_t, ((0, 0), (0, _NPAD - _N)),
                    constant_values=-1e30)
    anc = jnp.asarray(_ANCHORS_NP.T, jnp.float32)
    anc = jnp.pad(anc, ((0, 0), (0, _NPAD - _N)))

    out_shape = [
        jax.ShapeDtypeStruct((_B, 1, _LANE), jnp.int32),
        jax.ShapeDtypeStruct((_B, 4, _LANE), jnp.float32),
        jax.ShapeDtypeStruct((_B, 1, _LANE), jnp.float32),
        jax.ShapeDtypeStruct((_B, 1, _LANE), jnp.int32),
    ]
    vd, bx, sc, cl = pl.pallas_call(
        _nms_body,
        out_shape=out_shape,
        scratch_shapes=[
            pltpu.VMEM((8, _NPAD), jnp.float32),
        ],
    )(box_t, cls_t, anc)

    valid_detections = vd[:, 0, 0]
    nmsed_boxes = jnp.transpose(bx, (0, 2, 1))[:, :_MAX_DET, :]
    nmsed_scores = sc[:, 0, :_MAX_DET]
    nmsed_classes = cl[:, 0, :_MAX_DET]
    return valid_detections, nmsed_boxes, nmsed_scores, nmsed_classes


# speedup vs baseline: 3.1251x; 1.0231x over previous
import numpy as np
import jax
import jax.numpy as jnp
from jax.experimental import pallas as pl
from jax.experimental.pallas import tpu as pltpu

_NUM_CLASSES = 4
_IMAGE_SHAPE = (256, 256)
_SCORE_THR = 0.05
_SIGMA = 0.05
_MAX_PER_CLASS = 100
_MAX_DET = 100

_B = 2
_LANE = 128
_NEG = -jnp.inf


def _gen_anchors(image_shape):
    aspect_ratios = [0.5, 1.0, 2.0]
    scales = [2.0 ** x for x in [0.0, 1.0 / 3.0, 2.0 / 3.0]]
    areas = [float(x) ** 2 for x in [32, 64, 128, 256, 512]]
    all_anchors = []
    for level, area in zip(range(3, 8), areas):
        stride = 2 ** level
        dims = []
        for ratio in aspect_ratios:
            h = np.sqrt(area / ratio)
            w = area / h
            for s in scales:
                dims.append([w * s, h * s])
        dims = np.asarray(dims, np.float32)
        fh = int(np.ceil(image_shape[0] / stride))
        fw = int(np.ceil(image_shape[1] / stride))
        cx = (np.arange(fw, dtype=np.float32) + 0.5) * stride
        cy = (np.arange(fh, dtype=np.float32) + 0.5) * stride
        cxg, cyg = np.meshgrid(cx, cy)
        centers = np.stack([cxg, cyg], axis=-1).reshape(-1, 1, 2)
        centers = np.tile(centers, (1, dims.shape[0], 1))
        d = np.tile(dims[None, :, :], (centers.shape[0], 1, 1))
        all_anchors.append(np.concatenate([centers, d], axis=-1).reshape(-1, 4))
    return np.concatenate(all_anchors, axis=0)


_ANCHORS_NP = _gen_anchors(_IMAGE_SHAPE)
_N = _ANCHORS_NP.shape[0]
_NPAD = ((_N + _LANE - 1) // _LANE) * _LANE


def _nms_body(box_ref, cls_ref, anc_ref,
              vd_ref, bx_ref, sc_ref, cl_ref, s_ref):
    f32 = jnp.float32
    rowid = jax.lax.broadcasted_iota(jnp.int32, (8, _NPAD), 0)
    li = jax.lax.broadcasted_iota(jnp.int32, (8, _NPAD), 1)
    ki = jax.lax.broadcasted_iota(jnp.int32, (8, _LANE), 1)

    acx = anc_ref[0:1, :]
    acy = anc_ref[1:2, :]
    aw = anc_ref[2:3, :]
    ah = anc_ref[3:4, :]

    coords = []
    for b in range(_B):
        tx = box_ref[b, 0:1, :]
        ty = box_ref[b, 1:2, :]
        tw = box_ref[b, 2:3, :]
        th = box_ref[b, 3:4, :]
        cx = tx * aw + acx
        cy = ty * ah + acy
        w = jnp.exp(tw) * aw
        h = jnp.exp(th) * ah
        x1 = cx - w / 2.0
        y1 = cy - h / 2.0
        x2 = cx + w / 2.0
        y2 = cy + h / 2.0
        ar = (x2 - x1) * (y2 - y1)
        coords.append((x1, y1, x2, y2, ar))

    def expand(v0, v1):
        return jnp.where(rowid < 4, v0, v1)

    x1a = expand(coords[0][0], coords[1][0])
    y1a = expand(coords[0][1], coords[1][1])
    x2a = expand(coords[0][2], coords[1][2])
    y2a = expand(coords[0][3], coords[1][3])
    areas8 = expand(coords[0][4], coords[1][4])

    scores0 = jax.nn.sigmoid(cls_ref[...])
    s_init = jnp.where(scores0 > _SCORE_THR, scores0, _NEG)
    s_ref[...] = s_init
    m_init = jnp.max(s_init, axis=1, keepdims=True)

    def batch_cls(b):
        best = scores0[4 * b: 4 * b + 1, :]
        bidx = jnp.zeros_like(best, dtype=jnp.int32)
        for c in range(1, 4):
            sc = scores0[4 * b + c: 4 * b + c + 1, :]
            bidx = jnp.where(sc > best, jnp.int32(c), bidx)
            best = jnp.maximum(sc, best)
        return bidx
    cls8i = jnp.where(rowid < 4,
                      jnp.broadcast_to(batch_cls(0), (8, _NPAD)),
                      jnp.broadcast_to(batch_cls(1), (8, _NPAD)))
    li4c = li * 4 + cls8i
    _BIGKEY = jnp.int32(_NPAD * 4)

    def step(t, carry):
        m, acc_s, ax1, ay1, ax2, ay2, acls, aval = carry
        s = s_ref[...]
        eqm = s == m
        idx2 = jnp.min(jnp.where(eqm, li4c, _BIGKEY), axis=1, keepdims=True)
        onehot = li4c == idx2
        bcl = (idx2 & 3).astype(f32)

        def fetch(v):
            return jnp.sum(jnp.where(onehot, v, 0.0), axis=1, keepdims=True)

        bx1 = fetch(x1a)
        by1 = fetch(y1a)
        bx2 = fetch(x2a)
        by2 = fetch(y2a)
        valid = m > _SCORE_THR
        validf = valid.astype(f32)

        xx1 = jnp.maximum(bx1, x1a)
        yy1 = jnp.maximum(by1, y1a)
        xx2 = jnp.minimum(bx2, x2a)
        yy2 = jnp.minimum(by2, y2a)
        inter = jnp.maximum(xx2 - xx1, 0.0) * jnp.maximum(yy2 - yy1, 0.0)
        a_i = (bx2 - bx1) * (by2 - by1)
        iou = inter / (a_i + areas8 - inter + 1e-8)
        weight = jnp.exp(-0.5 * iou * iou / _SIGMA)
        ns = s * weight
        keep = (ns > _SCORE_THR) & jnp.logical_not(onehot)
        s_new = jnp.where(keep, ns, _NEG)
        s_ref[...] = s_new
        m_next = jnp.max(s_new, axis=1, keepdims=True)

        colhot = (ki == t).astype(f32)
        ssel = jnp.where(valid, m, 0.0)
        acc_s = acc_s + colhot * (ssel * validf)
        ax1 = ax1 + colhot * (bx1 * validf)
        ay1 = ay1 + colhot * (by1 * validf)
        ax2 = ax2 + colhot * (bx2 * validf)
        ay2 = ay2 + colhot * (by2 * validf)
        acls = acls + colhot * (bcl * validf)
        aval = aval + colhot * validf
        return m_next, acc_s, ax1, ay1, ax2, ay2, acls, aval

    zeros8 = jnp.zeros((8, _LANE), f32)
    _, acc_s, ax1, ay1, ax2, ay2, acls, aval = jax.lax.fori_loop(
        0, _MAX_PER_CLASS, step,
        (m_init, zeros8, zeros8, zeros8, zeros8, zeros8, zeros8, zeros8),
        unroll=10)

    r_iota = jax.lax.broadcasted_iota(jnp.int32, (_LANE, _LANE), 0)
    c_iota = jax.lax.broadcasted_iota(jnp.int32, (_LANE, _LANE), 1)
    sut = (r_iota < c_iota).astype(f32)
    kf = c_iota.astype(f32)
    kl = jax.lax.broadcasted_iota(jnp.int32, (1, _LANE), 1)
    hiP = jax.lax.Precision.HIGHEST

    v8 = aval > 0.0
    pre = jnp.dot(aval, sut, preferred_element_type=f32)
    rt = jnp.sum(aval, axis=1, keepdims=True)
    offs_rows = []
    nvs = []
    for b in range(_B):
        acc0 = jnp.zeros((1, 1), f32)
        for r in range(4):
            offs_rows.append(acc0)
            acc0 = acc0 + rt[4 * b + r: 4 * b + r + 1, 0:1]
        nvs.append(acc0)
    offs = jnp.concatenate(offs_rows, axis=0)
    rank_c = jnp.where(v8, pre + offs, 999.0)

    s_top = jnp.where(v8, acc_s, -1.0)
    s_t = jnp.transpose(s_top)
    rank_rows = []
    for b in range(_B):
        for rj in range(4):
            srow = s_top[4 * b + rj: 4 * b + rj + 1, :]
            cnt_acc = None
            for ri in range(4):
                scol = s_t[:, 4 * b + ri: 4 * b + ri + 1]
                gt = scol > srow
                if ri == rj:
                    big = gt | ((scol == srow) & (r_iota < c_iota))
                elif ri < rj:
                    big = gt | (scol == srow)
                else:
                    big = gt
                cnt = jnp.sum(big.astype(f32), axis=0, keepdims=True)
                cnt_acc = cnt if cnt_acc is None else cnt_acc + cnt
            rank_rows.append(cnt_acc)
    rank_t = jnp.concatenate(rank_rows, axis=0)

    ranks_tr = jnp.transpose(jnp.concatenate([rank_c, rank_t], axis=0))

    for b in range(_B):
        keep6 = jnp.zeros((6, _LANE), f32)
        top6 = jnp.zeros((6, _LANE), f32)
        for j in range(4):
            r = 4 * b + j
            vals = jnp.concatenate(
                [ax1[r:r + 1], ay1[r:r + 1], ax2[r:r + 1], ay2[r:r + 1],
                 acc_s[r:r + 1], acls[r:r + 1]], axis=0)
            p_c = (ranks_tr[:, r:r + 1] == kf).astype(f32)
            p_t = (ranks_tr[:, 8 + r:8 + r + 1] == kf).astype(f32)
            keep6 = keep6 + jnp.dot(vals, p_c, precision=hiP,
                                    preferred_element_type=f32)
            top6 = top6 + jnp.dot(vals, p_t, precision=hiP,
                                  preferred_element_type=f32)
        ox1, oy1 = keep6[0:1], keep6[1:2]
        ox2, oy2 = keep6[2:3], keep6[3:4]
        osc, ocl = keep6[4:5], keep6[5:6]
        px1, py1 = top6[0:1], top6[1:2]
        px2, py2 = top6[2:3], top6[3:4]
        psc = top6[4:5]
        nv = nvs[b]

        def lane_val(vec, j):
            return jnp.sum(jnp.where(kl == j, vec, 0.0), axis=(0, 1),
                           keepdims=True)
        cc0 = lane_val(ocl, 0)
        cc1 = lane_val(ocl, 1)
        cc2 = lane_val(ocl, 2)
        cc3 = lane_val(ocl, 3)
        buggy = jnp.where(ocl == 0.0, cc0,
                          jnp.where(ocl == 1.0, cc1,
                                    jnp.where(ocl == 2.0, cc2, cc3)))

        use_keep = nv <= f32(_MAX_DET)
        fx1 = jnp.where(use_keep, ox1, px1)
        fy1 = jnp.where(use_keep, oy1, py1)
        fx2 = jnp.where(use_keep, ox2, px2)
        fy2 = jnp.where(use_keep, oy2, py2)
        fsc = jnp.where(use_keep, osc, psc)
        ckeep = jnp.where(kl < nv.astype(jnp.int32), ocl, -1.0)
        fcl = jnp.where(use_keep, ckeep, buggy)

        bx_ref[b] = jnp.concatenate([fx1, fy1, fx2, fy2], axis=0)
        sc_ref[b] = fsc
        cl_ref[b] = fcl.astype(jnp.int32)
        vd_ref[b] = jnp.broadcast_to(
            jnp.minimum(nv, f32(_MAX_DET)).astype(jnp.int32), (1, _LANE))


def kernel(predictions):
    p = predictions.astype(jnp.float32)
    box_t = jnp.transpose(p[:, :, :4], (0, 2, 1))
    box_t = jnp.pad(box_t, ((0, 0), (0, 0), (0, _NPAD - _N)))
    cls_t = jnp.transpose(p[:, :, 4:], (0, 2, 1)).reshape(8, _N)
    cls_t = jnp.pad(cls_t, ((0, 0), (0, _NPAD - _N)),
                    constant_values=-1e30)
    anc = jnp.asarray(_ANCHORS_NP.T, jnp.float32)
    anc = jnp.pad(anc, ((0, 0), (0, _NPAD - _N)))

    out_shape = [
        jax.ShapeDtypeStruct((_B, 1, _LANE), jnp.int32),
        jax.ShapeDtypeStruct((_B, 4, _LANE), jnp.float32),
        jax.ShapeDtypeStruct((_B, 1, _LANE), jnp.float32),
        jax.ShapeDtypeStruct((_B, 1, _LANE), jnp.int32),
    ]
    vd, bx, sc, cl = pl.pallas_call(
        _nms_body,
        out_shape=out_shape,
        scratch_shapes=[
            pltpu.VMEM((8, _NPAD), jnp.float32),
        ],
    )(box_t, cls_t, anc)

    valid_detections = vd[:, 0, 0]
    nmsed_boxes = jnp.transpose(bx, (0, 2, 1))[:, :_MAX_DET, :]
    nmsed_scores = sc[:, 0, :_MAX_DET]
    nmsed_classes = cl[:, 0, :_MAX_DET]
    return valid_detections, nmsed_boxes, nmsed_scores, nmsed_classes


# speedup vs baseline: 3.1515x; 1.0085x over previous
import numpy as np
import jax
import jax.numpy as jnp
from jax.experimental import pallas as pl
from jax.experimental.pallas import tpu as pltpu

_NUM_CLASSES = 4
_IMAGE_SHAPE = (256, 256)
_SCORE_THR = 0.05
_SIGMA = 0.05
_MAX_PER_CLASS = 100
_MAX_DET = 100

_B = 2
_LANE = 128
_NEG = -jnp.inf


def _gen_anchors(image_shape):
    aspect_ratios = [0.5, 1.0, 2.0]
    scales = [2.0 ** x for x in [0.0, 1.0 / 3.0, 2.0 / 3.0]]
    areas = [float(x) ** 2 for x in [32, 64, 128, 256, 512]]
    all_anchors = []
    for level, area in zip(range(3, 8), areas):
        stride = 2 ** level
        dims = []
        for ratio in aspect_ratios:
            h = np.sqrt(area / ratio)
            w = area / h
            for s in scales:
                dims.append([w * s, h * s])
        dims = np.asarray(dims, np.float32)
        fh = int(np.ceil(image_shape[0] / stride))
        fw = int(np.ceil(image_shape[1] / stride))
        cx = (np.arange(fw, dtype=np.float32) + 0.5) * stride
        cy = (np.arange(fh, dtype=np.float32) + 0.5) * stride
        cxg, cyg = np.meshgrid(cx, cy)
        centers = np.stack([cxg, cyg], axis=-1).reshape(-1, 1, 2)
        centers = np.tile(centers, (1, dims.shape[0], 1))
        d = np.tile(dims[None, :, :], (centers.shape[0], 1, 1))
        all_anchors.append(np.concatenate([centers, d], axis=-1).reshape(-1, 4))
    return np.concatenate(all_anchors, axis=0)


_ANCHORS_NP = _gen_anchors(_IMAGE_SHAPE)
_N = _ANCHORS_NP.shape[0]
_NPAD = ((_N + _LANE - 1) // _LANE) * _LANE


def _nms_body(box_ref, cls_ref, anc_ref,
              vd_ref, bx_ref, sc_ref, cl_ref, s_ref):
    f32 = jnp.float32
    rowid = jax.lax.broadcasted_iota(jnp.int32, (8, _NPAD), 0)
    li = jax.lax.broadcasted_iota(jnp.int32, (8, _NPAD), 1)
    ki = jax.lax.broadcasted_iota(jnp.int32, (8, _LANE), 1)

    acx = anc_ref[0:1, :]
    acy = anc_ref[1:2, :]
    aw = anc_ref[2:3, :]
    ah = anc_ref[3:4, :]

    coords = []
    for b in range(_B):
        tx = box_ref[b, 0:1, :]
        ty = box_ref[b, 1:2, :]
        tw = box_ref[b, 2:3, :]
        th = box_ref[b, 3:4, :]
        cx = tx * aw + acx
        cy = ty * ah + acy
        w = jnp.exp(tw) * aw
        h = jnp.exp(th) * ah
        x1 = cx - w / 2.0
        y1 = cy - h / 2.0
        x2 = cx + w / 2.0
        y2 = cy + h / 2.0
        ar = (x2 - x1) * (y2 - y1)
        coords.append((x1, y1, x2, y2, ar))

    def expand(v0, v1):
        return jnp.where(rowid < 4, v0, v1)

    x1a = expand(coords[0][0], coords[1][0])
    y1a = expand(coords[0][1], coords[1][1])
    x2a = expand(coords[0][2], coords[1][2])
    y2a = expand(coords[0][3], coords[1][3])
    areas8 = expand(coords[0][4], coords[1][4])

    scores0 = jax.nn.sigmoid(cls_ref[...])
    s_init = jnp.where(scores0 > _SCORE_THR, scores0, _NEG)
    s_ref[...] = s_init
    m_init = jnp.max(s_init, axis=1, keepdims=True)

    def batch_cls(b):
        best = scores0[4 * b: 4 * b + 1, :]
        bidx = jnp.zeros_like(best, dtype=jnp.int32)
        for c in range(1, 4):
            sc = scores0[4 * b + c: 4 * b + c + 1, :]
            bidx = jnp.where(sc > best, jnp.int32(c), bidx)
            best = jnp.maximum(sc, best)
        return bidx
    cls8i = jnp.where(rowid < 4,
                      jnp.broadcast_to(batch_cls(0), (8, _NPAD)),
                      jnp.broadcast_to(batch_cls(1), (8, _NPAD)))
    li4c = li * 4 + cls8i
    _BIGKEY = jnp.int32(_NPAD * 4)

    def step(t, carry):
        m, acc_s, ax1, ay1, ax2, ay2, acls, aval = carry
        s = s_ref[...]
        eqm = s == m
        idx2 = jnp.min(jnp.where(eqm, li4c, _BIGKEY), axis=1, keepdims=True)
        onehot = li4c == idx2
        bcl = (idx2 & 3).astype(f32)

        def fetch(v):
            return jnp.sum(jnp.where(onehot, v, 0.0), axis=1, keepdims=True)

        bx1 = fetch(x1a)
        by1 = fetch(y1a)
        bx2 = fetch(x2a)
        by2 = fetch(y2a)
        valid = m > _SCORE_THR
        validf = valid.astype(f32)

        xx1 = jnp.maximum(bx1, x1a)
        yy1 = jnp.maximum(by1, y1a)
        xx2 = jnp.minimum(bx2, x2a)
        yy2 = jnp.minimum(by2, y2a)
        inter = jnp.maximum(xx2 - xx1, 0.0) * jnp.maximum(yy2 - yy1, 0.0)
        a_i = (bx2 - bx1) * (by2 - by1)
        iou = inter / (a_i + areas8 - inter + 1e-8)
        weight = jnp.exp(-0.5 * iou * iou / _SIGMA)
        ns = s * weight
        keep = (ns > _SCORE_THR) & jnp.logical_not(onehot)
        s_new = jnp.where(keep, ns, _NEG)
        s_ref[...] = s_new
        m_next = jnp.max(s_new, axis=1, keepdims=True)

        colhot = (ki == t).astype(f32)
        ssel = jnp.where(valid, m, 0.0)
        acc_s = acc_s + colhot * (ssel * validf)
        ax1 = ax1 + colhot * (bx1 * validf)
        ay1 = ay1 + colhot * (by1 * validf)
        ax2 = ax2 + colhot * (bx2 * validf)
        ay2 = ay2 + colhot * (by2 * validf)
        acls = acls + colhot * (bcl * validf)
        aval = aval + colhot * validf
        return m_next, acc_s, ax1, ay1, ax2, ay2, acls, aval

    zeros8 = jnp.zeros((8, _LANE), f32)
    _, acc_s, ax1, ay1, ax2, ay2, acls, aval = jax.lax.fori_loop(
        0, _MAX_PER_CLASS, step,
        (m_init, zeros8, zeros8, zeros8, zeros8, zeros8, zeros8, zeros8),
        unroll=25)

    r_iota = jax.lax.broadcasted_iota(jnp.int32, (_LANE, _LANE), 0)
    c_iota = jax.lax.broadcasted_iota(jnp.int32, (_LANE, _LANE), 1)
    sut = (r_iota < c_iota).astype(f32)
    kf = c_iota.astype(f32)
    kl = jax.lax.broadcasted_iota(jnp.int32, (1, _LANE), 1)
    hiP = jax.lax.Precision.HIGHEST

    v8 = aval > 0.0
    pre = jnp.dot(aval, sut, preferred_element_type=f32)
    rt = jnp.sum(aval, axis=1, keepdims=True)
    offs_rows = []
    nvs = []
    for b in range(_B):
        acc0 = jnp.zeros((1, 1), f32)
        for r in range(4):
            offs_rows.append(acc0)
            acc0 = acc0 + rt[4 * b + r: 4 * b + r + 1, 0:1]
        nvs.append(acc0)
    offs = jnp.concatenate(offs_rows, axis=0)
    rank_c = jnp.where(v8, pre + offs, 999.0)

    s_top = jnp.where(v8, acc_s, -1.0)
    s_t = jnp.transpose(s_top)
    rank_rows = []
    for b in range(_B):
        for rj in range(4):
            srow = s_top[4 * b + rj: 4 * b + rj + 1, :]
            cnt_acc = None
            for ri in range(4):
                scol = s_t[:, 4 * b + ri: 4 * b + ri + 1]
                gt = scol > srow
                if ri == rj:
                    big = gt | ((scol == srow) & (r_iota < c_iota))
                elif ri < rj:
                    big = gt | (scol == srow)
                else:
                    big = gt
                cnt = jnp.sum(big.astype(f32), axis=0, keepdims=True)
                cnt_acc = cnt if cnt_acc is None else cnt_acc + cnt
            rank_rows.append(cnt_acc)
    rank_t = jnp.concatenate(rank_rows, axis=0)

    ranks_tr = jnp.transpose(jnp.concatenate([rank_c, rank_t], axis=0))

    for b in range(_B):
        keep6 = jnp.zeros((6, _LANE), f32)
        top6 = jnp.zeros((6, _LANE), f32)
        for j in range(4):
            r = 4 * b + j
            vals = jnp.concatenate(
                [ax1[r:r + 1], ay1[r:r + 1], ax2[r:r + 1], ay2[r:r + 1],
                 acc_s[r:r + 1], acls[r:r + 1]], axis=0)
            p_c = (ranks_tr[:, r:r + 1] == kf).astype(f32)
            p_t = (ranks_tr[:, 8 + r:8 + r + 1] == kf).astype(f32)
            keep6 = keep6 + jnp.dot(vals, p_c, precision=hiP,
                                    preferred_element_type=f32)
            top6 = top6 + jnp.dot(vals, p_t, precision=hiP,
                                  preferred_element_type=f32)
        ox1, oy1 = keep6[0:1], keep6[1:2]
        ox2, oy2 = keep6[2:3], keep6[3:4]
        osc, ocl = keep6[4:5], keep6[5:6]
        px1, py1 = top6[0:1], top6[1:2]
        px2, py2 = top6[2:3], top6[3:4]
        psc = top6[4:5]
        nv = nvs[b]

        def lane_val(vec, j):
            return jnp.sum(jnp.where(kl == j, vec, 0.0), axis=(0, 1),
                           keepdims=True)
        cc0 = lane_val(ocl, 0)
        cc1 = lane_val(ocl, 1)
        cc2 = lane_val(ocl, 2)
        cc3 = lane_val(ocl, 3)
        buggy = jnp.where(ocl == 0.0, cc0,
                          jnp.where(ocl == 1.0, cc1,
                                    jnp.where(ocl == 2.0, cc2, cc3)))

        use_keep = nv <= f32(_MAX_DET)
        fx1 = jnp.where(use_keep, ox1, px1)
        fy1 = jnp.where(use_keep, oy1, py1)
        fx2 = jnp.where(use_keep, ox2, px2)
        fy2 = jnp.where(use_keep, oy2, py2)
        fsc = jnp.where(use_keep, osc, psc)
        ckeep = jnp.where(kl < nv.astype(jnp.int32), ocl, -1.0)
        fcl = jnp.where(use_keep, ckeep, buggy)

        bx_ref[b] = jnp.concatenate([fx1, fy1, fx2, fy2], axis=0)
        sc_ref[b] = fsc
        cl_ref[b] = fcl.astype(jnp.int32)
        vd_ref[b] = jnp.broadcast_to(
            jnp.minimum(nv, f32(_MAX_DET)).astype(jnp.int32), (1, _LANE))


def kernel(predictions):
    p = predictions.astype(jnp.float32)
    box_t = jnp.transpose(p[:, :, :4], (0, 2, 1))
    box_t = jnp.pad(box_t, ((0, 0), (0, 0), (0, _NPAD - _N)))
    cls_t = jnp.transpose(p[:, :, 4:], (0, 2, 1)).reshape(8, _N)
    cls_t = jnp.pad(cls_t, ((0, 0), (0, _NPAD - _N)),
                    constant_values=-1e30)
    anc = jnp.asarray(_ANCHORS_NP.T, jnp.float32)
    anc = jnp.pad(anc, ((0, 0), (0, _NPAD - _N)))

    out_shape = [
        jax.ShapeDtypeStruct((_B, 1, _LANE), jnp.int32),
        jax.ShapeDtypeStruct((_B, 4, _LANE), jnp.float32),
        jax.ShapeDtypeStruct((_B, 1, _LANE), jnp.float32),
        jax.ShapeDtypeStruct((_B, 1, _LANE), jnp.int32),
    ]
    vd, bx, sc, cl = pl.pallas_call(
        _nms_body,
        out_shape=out_shape,
        scratch_shapes=[
            pltpu.VMEM((8, _NPAD), jnp.float32),
        ],
    )(box_t, cls_t, anc)

    valid_detections = vd[:, 0, 0]
    nmsed_boxes = jnp.transpose(bx, (0, 2, 1))[:, :_MAX_DET, :]
    nmsed_scores = sc[:, 0, :_MAX_DET]
    nmsed_classes = cl[:, 0, :_MAX_DET]
    return valid_detections, nmsed_boxes, nmsed_scores, nmsed_classes
